# Initial kernel scaffold; baseline (speedup 1.0000x reference)
import jax
import jax.numpy as jnp
from jax.experimental import pallas as pl


def kernel(x, edge_index, W1, b1, W2, b2, W3, b3):
    raise NotImplementedError("write your pallas kernel here")


# speedup vs baseline: 17.3387x; 17.3387x over previous
import functools

import jax
import jax.numpy as jnp
from jax import lax
from jax.experimental import pallas as pl
from jax.experimental.pallas import tpu as pltpu
from jax.experimental.pallas import tpu_sc as plsc

NC = 2
NS = 16
ROW_BLK = 1280


def _mesh():
    return plsc.VectorSubcoreMesh(core_axis_name="c", subcore_axis_name="s")


_SC_PARAMS = pltpu.CompilerParams(use_tc_tiling_on_sc=False)


def _make_deg_count(n, e_pad):
    per_tile = e_pad // (NC * NS)
    k = 1024
    assert per_tile % k == 0 and n % (NS * ROW_BLK) == 0
    iters = per_tile // k
    rows_per = n // NS
    n_copies = rows_per // ROW_BLK

    @functools.partial(
        pl.kernel,
        out_type=jax.ShapeDtypeStruct((NC, n, 4), jnp.float32),
        mesh=_mesh(),
        compiler_params=_SC_PARAMS,
        scratch_types=[
            pltpu.VMEM((8, 128), jnp.int32),
            pltpu.VMEM((128, 4), jnp.float32),
            pltpu.VMEM((ROW_BLK, 4), jnp.float32),
            pltpu.VMEM_SHARED((n, 4), jnp.float32),
            pltpu.SemaphoreType.DMA,
        ],
    )
    def deg_kernel(dst_hbm, ones_hbm, zrows, out, dst_v, ones_v, zb, acc, sem):
        c = lax.axis_index("c")
        s = lax.axis_index("s")
        g = c * NS + s
        pltpu.sync_copy(ones_hbm, ones_v)
        pltpu.sync_copy(zrows, zb)
        rbase = s * rows_per
        for z in range(n_copies):
            pltpu.sync_copy(zb, acc.at[pl.ds(rbase + z * ROW_BLK, ROW_BLK)])
        plsc.subcore_barrier()
        row0 = g * (per_tile // 128)

        @pl.loop(0, iters)
        def _(i):
            pltpu.sync_copy(dst_hbm.at[pl.ds(row0 + i * (k // 128), k // 128)], dst_v)
            for j in range(k // 128):
                pltpu.sync_copy(ones_v, acc.at[dst_v.at[j]], add=True)

        plsc.subcore_barrier()
        for z in range(n_copies):
            sl = pl.ds(rbase + z * ROW_BLK, ROW_BLK)
            pltpu.sync_copy(acc.at[sl], out.at[c].at[sl])

    return deg_kernel


def _make_segsum8(n, e_pad):
    per_tile = e_pad // (NC * NS)
    k = 1024
    assert per_tile % k == 0
    iters = per_tile // k
    rows_per = n // NS
    n_copies = rows_per // ROW_BLK

    @functools.partial(
        pl.kernel,
        out_type=jax.ShapeDtypeStruct((NC, n, 8), jnp.float32),
        mesh=_mesh(),
        compiler_params=_SC_PARAMS,
        scratch_types=[
            pltpu.VMEM((k,), jnp.int32),
            pltpu.VMEM((8, 128), jnp.int32),
            pltpu.VMEM((k, 8), jnp.float32),
            pltpu.VMEM((ROW_BLK, 8), jnp.float32),
            pltpu.VMEM_SHARED((n, 8), jnp.float32),
            pltpu.SemaphoreType.DMA,
        ],
    )
    def segsum_kernel(table, src_hbm, dst_hbm, zrows, out, src_v, dst_v, rows_v, zb, acc, sem):
        c = lax.axis_index("c")
        s = lax.axis_index("s")
        g = c * NS + s
        pltpu.sync_copy(zrows, zb)
        rbase = s * rows_per
        for z in range(n_copies):
            pltpu.sync_copy(zb, acc.at[pl.ds(rbase + z * ROW_BLK, ROW_BLK)])
        plsc.subcore_barrier()
        ebase = g * per_tile
        row0 = g * (per_tile // 128)

        @pl.loop(0, iters)
        def _(i):
            pltpu.sync_copy(src_hbm.at[pl.ds(ebase + i * k, k)], src_v)
            pltpu.sync_copy(dst_hbm.at[pl.ds(row0 + i * (k // 128), k // 128)], dst_v)
            pltpu.async_copy(table.at[src_v], rows_v, sem).wait()
            for j in range(k // 128):
                pltpu.sync_copy(rows_v.at[pl.ds(j * 128, 128)], acc.at[dst_v.at[j]], add=True)

        plsc.subcore_barrier()
        for z in range(n_copies):
            sl = pl.ds(rbase + z * ROW_BLK, ROW_BLK)
            pltpu.sync_copy(acc.at[sl], out.at[c].at[sl])

    return segsum_kernel


def _make_segsum8x8(n, e_pad):
    per_tile = e_pad // NS
    k = 2048
    assert per_tile % k == 0
    iters = per_tile // k
    rows_per = n // NS
    n_copies = rows_per // ROW_BLK

    @functools.partial(
        pl.kernel,
        out_type=jax.ShapeDtypeStruct((8, n, 8), jnp.float32),
        mesh=_mesh(),
        compiler_params=_SC_PARAMS,
        scratch_types=[
            pltpu.VMEM((k,), jnp.int32),
            pltpu.VMEM((16, 128), jnp.int32),
            pltpu.VMEM((k, 8), jnp.float32),
            pltpu.VMEM((ROW_BLK, 8), jnp.float32),
            pltpu.VMEM_SHARED((n, 8), jnp.float32),
            pltpu.SemaphoreType.DMA,
        ],
    )
    def segsum_kernel(table, src_hbm, dst_hbm, zrows, out, src_v, dst_v, rows_v, zb, acc, sem):
        c = lax.axis_index("c")
        s = lax.axis_index("s")
        pltpu.sync_copy(zrows, zb)
        rbase = s * rows_per
        ebase = s * per_tile
        row0 = s * (per_tile // 128)
        for cc in range(4):
            cid = c * 4 + cc
            for z in range(n_copies):
                pltpu.sync_copy(zb, acc.at[pl.ds(rbase + z * ROW_BLK, ROW_BLK)])
            plsc.subcore_barrier()

            @pl.loop(0, iters)
            def _(i):
                pltpu.sync_copy(src_hbm.at[pl.ds(ebase + i * k, k)], src_v)
                pltpu.sync_copy(dst_hbm.at[pl.ds(row0 + i * (k // 128), k // 128)], dst_v)
                pltpu.async_copy(table.at[cid].at[src_v], rows_v, sem).wait()
                for j in range(k // 128):
                    pltpu.sync_copy(rows_v.at[pl.ds(j * 128, 128)], acc.at[dst_v.at[j]], add=True)

            plsc.subcore_barrier()
            for z in range(n_copies):
                sl = pl.ds(rbase + z * ROW_BLK, ROW_BLK)
                pltpu.sync_copy(acc.at[sl], out.at[cid].at[sl])
            plsc.subcore_barrier()

    return segsum_kernel


_R = 2000


def _row0_mask(pid, r):
    return (lax.broadcasted_iota(jnp.int32, (r, 1), 0) == 0) & (pid == 0)


def _make_tc1(n, pad_edges):
    def body(degp, xp, xhat, dinv):
        pid = pl.program_id(0)
        d = degp[...]
        deg = d[0, :, 0:1] + d[1, :, 0:1] + 1.0
        deg = deg - jnp.where(_row0_mask(pid, _R), float(pad_edges), 0.0)
        di = lax.rsqrt(deg)
        dinv[...] = di
        xhat[...] = xp[...] * di

    return pl.pallas_call(
        body,
        grid=(n // _R,),
        in_specs=[
            pl.BlockSpec((NC, _R, 4), lambda i: (0, i, 0)),
            pl.BlockSpec((_R, 8), lambda i: (i, 0)),
        ],
        out_specs=[
            pl.BlockSpec((_R, 8), lambda i: (i, 0)),
            pl.BlockSpec((_R, 1), lambda i: (i, 0)),
        ],
        out_shape=[
            jax.ShapeDtypeStruct((n, 8), jnp.float32),
            jax.ShapeDtypeStruct((n, 1), jnp.float32),
        ],
    )


def _make_tc2(n, pad_edges):
    def body(s1p, xhat, dinv, w1, b1, out):
        pid = pl.program_id(0)
        sp = s1p[...]
        xh = xhat[...]
        s = sp[0] + sp[1]
        s = s - jnp.where(_row0_mask(pid, _R), float(pad_edges) * xh[0:1, :], 0.0)
        a1 = dinv[...] * (s + xh)
        h1 = jnp.dot(a1, w1[...], preferred_element_type=jnp.float32,
                     precision=lax.Precision.HIGHEST) + b1[...]
        out[...] = jnp.maximum(h1, 0.0) * dinv[...]

    return pl.pallas_call(
        body,
        grid=(n // _R,),
        in_specs=[
            pl.BlockSpec((NC, _R, 8), lambda i: (0, i, 0)),
            pl.BlockSpec((_R, 8), lambda i: (i, 0)),
            pl.BlockSpec((_R, 1), lambda i: (i, 0)),
            pl.BlockSpec((8, 64), lambda i: (0, 0)),
            pl.BlockSpec((1, 64), lambda i: (0, 0)),
        ],
        out_specs=pl.BlockSpec((_R, 64), lambda i: (i, 0)),
        out_shape=jax.ShapeDtypeStruct((n, 64), jnp.float32),
    )


def _make_tc3(n, pad_edges):
    def body(s2, h1hat, dinv, w2, b2, w3p, out):
        pid = pl.program_id(0)
        hh = h1hat[...]
        s = s2[...]
        s = s - jnp.where(_row0_mask(pid, _R), float(pad_edges) * hh[0:1, :], 0.0)
        a2 = dinv[...] * (s + hh)
        h2 = jnp.dot(a2, w2[...], preferred_element_type=jnp.float32,
                     precision=lax.Precision.HIGHEST) + b2[...]
        h2 = jnp.maximum(h2, 0.0)
        t = jnp.dot(h2, w3p[...], preferred_element_type=jnp.float32,
                    precision=lax.Precision.HIGHEST)
        out[...] = t * dinv[...]

    return pl.pallas_call(
        body,
        grid=(n // _R,),
        in_specs=[
            pl.BlockSpec((_R, 64), lambda i: (i, 0)),
            pl.BlockSpec((_R, 64), lambda i: (i, 0)),
            pl.BlockSpec((_R, 1), lambda i: (i, 0)),
            pl.BlockSpec((64, 64), lambda i: (0, 0)),
            pl.BlockSpec((1, 64), lambda i: (0, 0)),
            pl.BlockSpec((64, 8), lambda i: (0, 0)),
        ],
        out_specs=pl.BlockSpec((_R, 8), lambda i: (i, 0)),
        out_shape=jax.ShapeDtypeStruct((n, 8), jnp.float32),
    )


def _make_tc4(n, pad_edges):
    def body(s3p, h3hat, dinv, b3, out):
        pid = pl.program_id(0)
        sp = s3p[...]
        hh = h3hat[...]
        s = sp[0] + sp[1]
        s = s - jnp.where(_row0_mask(pid, _R), float(pad_edges) * hh[0:1, :], 0.0)
        o = dinv[...] * (s + hh)
        out[...] = o[:, 0:1] + b3[...]

    return pl.pallas_call(
        body,
        grid=(n // _R,),
        in_specs=[
            pl.BlockSpec((NC, _R, 8), lambda i: (0, i, 0)),
            pl.BlockSpec((_R, 8), lambda i: (i, 0)),
            pl.BlockSpec((_R, 1), lambda i: (i, 0)),
            pl.BlockSpec((1, 1), lambda i: (0, 0)),
        ],
        out_specs=pl.BlockSpec((_R, 1), lambda i: (i, 0)),
        out_shape=jax.ShapeDtypeStruct((n, 1), jnp.float32),
    )


def kernel(x, edge_index, W1, b1, W2, b2, W3, b3):
    n, f = x.shape
    e = edge_index.shape[1]
    h = W1.shape[1]
    assert f == 5 and h == 64 and W3.shape[1] == 1

    unit = NC * NS * 1024
    e_pad = ((e + unit - 1) // unit) * unit
    pad = e_pad - e
    n_unit = NS * ROW_BLK
    n_pad = ((n + n_unit - 1) // n_unit) * n_unit

    src = jnp.concatenate([edge_index[0], jnp.zeros((pad,), jnp.int32)])
    dst = jnp.concatenate([edge_index[1], jnp.zeros((pad,), jnp.int32)])
    dst2d = dst.reshape(-1, 128)
    ones = jnp.ones((128, 4), jnp.float32)
    z4 = jnp.zeros((ROW_BLK, 4), jnp.float32)
    z8 = jnp.zeros((ROW_BLK, 8), jnp.float32)

    degp = _make_deg_count(n_pad, e_pad)(dst2d, ones, z4)
    xpad = jnp.pad(x, ((0, 0), (0, 8 - f)))
    xhat, dinv = _make_tc1(n, pad)(degp, xpad)

    segsum8 = _make_segsum8(n_pad, e_pad)
    s1p = segsum8(xhat, src, dst2d, z8)
    w1p = jnp.pad(W1, ((0, 8 - f), (0, 0)))
    h1hat = _make_tc2(n, pad)(s1p, xhat, dinv, w1p, b1.reshape(1, h))

    t8 = h1hat.reshape(n, 8, 8).transpose(1, 0, 2)
    s2c = _make_segsum8x8(n_pad, e_pad)(t8, src, dst2d, z8)
    s2 = s2c.transpose(1, 0, 2).reshape(n_pad, h)

    w3p = jnp.pad(W3, ((0, 0), (0, 7)))
    h3hatp = _make_tc3(n, pad)(s2, h1hat, dinv, W2, b2.reshape(1, h), w3p)

    s3p = segsum8(h3hatp, src, dst2d, z8)
    out = _make_tc4(n, pad)(s3p, h3hatp, dinv, b3.reshape(1, 1))
    return out


# speedup vs baseline: 22.8886x; 1.3201x over previous
import functools

import jax
import jax.numpy as jnp
from jax import lax
from jax.experimental import pallas as pl
from jax.experimental.pallas import tpu as pltpu
from jax.experimental.pallas import tpu_sc as plsc

NC = 2
NS = 16
ROW_BLK = 1280


def _mesh():
    return plsc.VectorSubcoreMesh(core_axis_name="c", subcore_axis_name="s")


_SC_PARAMS = pltpu.CompilerParams(use_tc_tiling_on_sc=False)


def _make_deg_count(n, e_pad):
    per_tile = e_pad // (NC * NS)
    k = 1024
    assert per_tile % k == 0 and n % (NS * ROW_BLK) == 0
    iters = per_tile // k
    rows_per = n // NS
    n_copies = rows_per // ROW_BLK

    @functools.partial(
        pl.kernel,
        out_type=jax.ShapeDtypeStruct((NC, n, 4), jnp.float32),
        mesh=_mesh(),
        compiler_params=_SC_PARAMS,
        scratch_types=[
            pltpu.VMEM((8, 128), jnp.int32),
            pltpu.VMEM((128, 4), jnp.float32),
            pltpu.VMEM((ROW_BLK, 4), jnp.float32),
            pltpu.VMEM_SHARED((n, 4), jnp.float32),
            pltpu.SemaphoreType.DMA,
        ],
    )
    def deg_kernel(dst_hbm, ones_hbm, zrows, out, dst_v, ones_v, zb, acc, sem):
        c = lax.axis_index("c")
        s = lax.axis_index("s")
        g = c * NS + s
        pltpu.sync_copy(ones_hbm, ones_v)
        pltpu.sync_copy(zrows, zb)
        rbase = s * rows_per
        for z in range(n_copies):
            pltpu.sync_copy(zb, acc.at[pl.ds(rbase + z * ROW_BLK, ROW_BLK)])
        plsc.subcore_barrier()
        row0 = g * (per_tile // 128)

        @pl.loop(0, iters)
        def _(i):
            pltpu.sync_copy(dst_hbm.at[pl.ds(row0 + i * (k // 128), k // 128)], dst_v)
            for j in range(k // 128):
                pltpu.sync_copy(ones_v, acc.at[dst_v.at[j]], add=True)

        plsc.subcore_barrier()
        for z in range(n_copies):
            sl = pl.ds(rbase + z * ROW_BLK, ROW_BLK)
            pltpu.sync_copy(acc.at[sl], out.at[c].at[sl])

    return deg_kernel


def _sweep_pipelined(table2d, src_hbm, dst_hbm, acc, srcb, dstb, rowsb, gsems,
                     ssem, ebase, row0, iters, k, m):
    def load(it, b):
        pltpu.sync_copy(src_hbm.at[pl.ds(ebase + it * k, k)], srcb[b])
        pltpu.sync_copy(dst_hbm.at[pl.ds(row0 + it * m, m)], dstb[b])

    def gather(b):
        return pltpu.async_copy(table2d.at[srcb[b]], rowsb[b], gsems[b])

    def scatter(b):
        ds_ = [
            pltpu.async_copy(rowsb[b].at[pl.ds(j * 128, 128)],
                             acc.at[dstb[b].at[j]], ssem, add=True)
            for j in range(m)
        ]
        for d in ds_:
            d.wait()

    load(0, 0)
    gather(0)

    @pl.loop(0, iters, step=2)
    def _(i):
        load(i + 1, 1)
        g1 = gather(1)
        pltpu.make_async_copy(table2d.at[srcb[0]], rowsb[0], gsems[0]).wait()
        scatter(0)

        @pl.when(i + 2 < iters)
        def _():
            load(i + 2, 0)
            gather(0)

        g1.wait()
        scatter(1)


def _make_segsum8(n, e_pad):
    per_tile = e_pad // (NC * NS)
    k = 1792
    m = k // 128
    assert per_tile % k == 0 and (per_tile // k) % 2 == 0
    iters = per_tile // k
    rows_per = n // NS
    n_copies = rows_per // ROW_BLK

    @functools.partial(
        pl.kernel,
        out_type=jax.ShapeDtypeStruct((NC, n, 8), jnp.float32),
        mesh=_mesh(),
        compiler_params=_SC_PARAMS,
        scratch_types=[
            pltpu.VMEM((k,), jnp.int32), pltpu.VMEM((k,), jnp.int32),
            pltpu.VMEM((m, 128), jnp.int32), pltpu.VMEM((m, 128), jnp.int32),
            pltpu.VMEM((k, 8), jnp.float32), pltpu.VMEM((k, 8), jnp.float32),
            pltpu.VMEM((ROW_BLK, 8), jnp.float32),
            pltpu.VMEM_SHARED((n, 8), jnp.float32),
            pltpu.SemaphoreType.DMA, pltpu.SemaphoreType.DMA,
            pltpu.SemaphoreType.DMA,
        ],
    )
    def segsum_kernel(table, src_hbm, dst_hbm, zrows, out,
                      s0, s1, d0, d1, r0, r1, zb, acc, gs0, gs1, ss):
        c = lax.axis_index("c")
        s = lax.axis_index("s")
        g = c * NS + s
        pltpu.sync_copy(zrows, zb)
        rbase = s * rows_per
        for z in range(n_copies):
            pltpu.sync_copy(zb, acc.at[pl.ds(rbase + z * ROW_BLK, ROW_BLK)])
        plsc.subcore_barrier()
        _sweep_pipelined(table, src_hbm, dst_hbm, acc, (s0, s1), (d0, d1),
                         (r0, r1), (gs0, gs1), ss, g * per_tile,
                         g * (per_tile // 128), iters, k, m)
        plsc.subcore_barrier()
        for z in range(n_copies):
            sl = pl.ds(rbase + z * ROW_BLK, ROW_BLK)
            pltpu.sync_copy(acc.at[sl], out.at[c].at[sl])

    return segsum_kernel


def _make_segsum8x8(n, e_pad):
    per_tile = e_pad // NS
    k = 1792
    m = k // 128
    assert per_tile % k == 0 and (per_tile // k) % 2 == 0
    iters = per_tile // k
    rows_per = n // NS
    n_copies = rows_per // ROW_BLK

    @functools.partial(
        pl.kernel,
        out_type=jax.ShapeDtypeStruct((8, n, 8), jnp.float32),
        mesh=_mesh(),
        compiler_params=_SC_PARAMS,
        scratch_types=[
            pltpu.VMEM((k,), jnp.int32), pltpu.VMEM((k,), jnp.int32),
            pltpu.VMEM((m, 128), jnp.int32), pltpu.VMEM((m, 128), jnp.int32),
            pltpu.VMEM((k, 8), jnp.float32), pltpu.VMEM((k, 8), jnp.float32),
            pltpu.VMEM((ROW_BLK, 8), jnp.float32),
            pltpu.VMEM_SHARED((n, 8), jnp.float32),
            pltpu.SemaphoreType.DMA, pltpu.SemaphoreType.DMA,
            pltpu.SemaphoreType.DMA,
        ],
    )
    def segsum_kernel(table, src_hbm, dst_hbm, zrows, out,
                      s0, s1, d0, d1, r0, r1, zb, acc, gs0, gs1, ss):
        c = lax.axis_index("c")
        s = lax.axis_index("s")
        pltpu.sync_copy(zrows, zb)
        rbase = s * rows_per
        ebase = s * per_tile
        row0 = s * (per_tile // 128)
        for cc in range(4):
            cid = c * 4 + cc
            for z in range(n_copies):
                pltpu.sync_copy(zb, acc.at[pl.ds(rbase + z * ROW_BLK, ROW_BLK)])
            plsc.subcore_barrier()
            _sweep_pipelined(table.at[cid], src_hbm, dst_hbm, acc, (s0, s1),
                             (d0, d1), (r0, r1), (gs0, gs1), ss, ebase, row0,
                             iters, k, m)
            plsc.subcore_barrier()
            for z in range(n_copies):
                sl = pl.ds(rbase + z * ROW_BLK, ROW_BLK)
                pltpu.sync_copy(acc.at[sl], out.at[cid].at[sl])
            plsc.subcore_barrier()

    return segsum_kernel


_R = 2000


def _row0_mask(pid, r):
    return (lax.broadcasted_iota(jnp.int32, (r, 1), 0) == 0) & (pid == 0)


def _make_tc1(n, pad_edges):
    def body(degp, xp, xhat, dinv):
        pid = pl.program_id(0)
        d = degp[...]
        deg = d[0, :, 0:1] + d[1, :, 0:1] + 1.0
        deg = deg - jnp.where(_row0_mask(pid, _R), float(pad_edges), 0.0)
        di = lax.rsqrt(deg)
        dinv[...] = di
        xhat[...] = xp[...] * di

    return pl.pallas_call(
        body,
        grid=(n // _R,),
        in_specs=[
            pl.BlockSpec((NC, _R, 4), lambda i: (0, i, 0)),
            pl.BlockSpec((_R, 8), lambda i: (i, 0)),
        ],
        out_specs=[
            pl.BlockSpec((_R, 8), lambda i: (i, 0)),
            pl.BlockSpec((_R, 1), lambda i: (i, 0)),
        ],
        out_shape=[
            jax.ShapeDtypeStruct((n, 8), jnp.float32),
            jax.ShapeDtypeStruct((n, 1), jnp.float32),
        ],
    )


def _make_tc2(n, pad_edges):
    def body(s1p, xhat, dinv, w1, b1, out):
        pid = pl.program_id(0)
        sp = s1p[...]
        xh = xhat[...]
        s = sp[0] + sp[1]
        s = s - jnp.where(_row0_mask(pid, _R), float(pad_edges) * xh[0:1, :], 0.0)
        a1 = dinv[...] * (s + xh)
        h1 = jnp.dot(a1, w1[...], preferred_element_type=jnp.float32,
                     precision=lax.Precision.HIGHEST) + b1[...]
        out[...] = jnp.maximum(h1, 0.0) * dinv[...]

    return pl.pallas_call(
        body,
        grid=(n // _R,),
        in_specs=[
            pl.BlockSpec((NC, _R, 8), lambda i: (0, i, 0)),
            pl.BlockSpec((_R, 8), lambda i: (i, 0)),
            pl.BlockSpec((_R, 1), lambda i: (i, 0)),
            pl.BlockSpec((8, 64), lambda i: (0, 0)),
            pl.BlockSpec((1, 64), lambda i: (0, 0)),
        ],
        out_specs=pl.BlockSpec((_R, 64), lambda i: (i, 0)),
        out_shape=jax.ShapeDtypeStruct((n, 64), jnp.float32),
    )


def _make_tc3(n, pad_edges):
    def body(s2, h1hat, dinv, w2, b2, w3p, out):
        pid = pl.program_id(0)
        hh = h1hat[...]
        s = s2[...]
        s = s - jnp.where(_row0_mask(pid, _R), float(pad_edges) * hh[0:1, :], 0.0)
        a2 = dinv[...] * (s + hh)
        h2 = jnp.dot(a2, w2[...], preferred_element_type=jnp.float32,
                     precision=lax.Precision.HIGHEST) + b2[...]
        h2 = jnp.maximum(h2, 0.0)
        t = jnp.dot(h2, w3p[...], preferred_element_type=jnp.float32,
                    precision=lax.Precision.HIGHEST)
        out[...] = t * dinv[...]

    return pl.pallas_call(
        body,
        grid=(n // _R,),
        in_specs=[
            pl.BlockSpec((_R, 64), lambda i: (i, 0)),
            pl.BlockSpec((_R, 64), lambda i: (i, 0)),
            pl.BlockSpec((_R, 1), lambda i: (i, 0)),
            pl.BlockSpec((64, 64), lambda i: (0, 0)),
            pl.BlockSpec((1, 64), lambda i: (0, 0)),
            pl.BlockSpec((64, 8), lambda i: (0, 0)),
        ],
        out_specs=pl.BlockSpec((_R, 8), lambda i: (i, 0)),
        out_shape=jax.ShapeDtypeStruct((n, 8), jnp.float32),
    )


def _make_tc4(n, pad_edges):
    def body(s3p, h3hat, dinv, b3, out):
        pid = pl.program_id(0)
        sp = s3p[...]
        hh = h3hat[...]
        s = sp[0] + sp[1]
        s = s - jnp.where(_row0_mask(pid, _R), float(pad_edges) * hh[0:1, :], 0.0)
        o = dinv[...] * (s + hh)
        out[...] = o[:, 0:1] + b3[...]

    return pl.pallas_call(
        body,
        grid=(n // _R,),
        in_specs=[
            pl.BlockSpec((NC, _R, 8), lambda i: (0, i, 0)),
            pl.BlockSpec((_R, 8), lambda i: (i, 0)),
            pl.BlockSpec((_R, 1), lambda i: (i, 0)),
            pl.BlockSpec((1, 1), lambda i: (0, 0)),
        ],
        out_specs=pl.BlockSpec((_R, 1), lambda i: (i, 0)),
        out_shape=jax.ShapeDtypeStruct((n, 1), jnp.float32),
    )


def kernel(x, edge_index, W1, b1, W2, b2, W3, b3):
    n, f = x.shape
    e = edge_index.shape[1]
    h = W1.shape[1]
    assert f == 5 and h == 64 and W3.shape[1] == 1

    unit = NC * NS * 1024
    e_pad = ((e + unit - 1) // unit) * unit
    pad = e_pad - e
    n_unit = NS * ROW_BLK
    n_pad = ((n + n_unit - 1) // n_unit) * n_unit

    src = jnp.concatenate([edge_index[0], jnp.zeros((pad,), jnp.int32)])
    dst = jnp.concatenate([edge_index[1], jnp.zeros((pad,), jnp.int32)])
    dst2d = dst.reshape(-1, 128)
    ones = jnp.ones((128, 4), jnp.float32)
    z4 = jnp.zeros((ROW_BLK, 4), jnp.float32)
    z8 = jnp.zeros((ROW_BLK, 8), jnp.float32)

    degp = _make_deg_count(n_pad, e_pad)(dst2d, ones, z4)
    xpad = jnp.pad(x, ((0, 0), (0, 8 - f)))
    xhat, dinv = _make_tc1(n, pad)(degp, xpad)

    segsum8 = _make_segsum8(n_pad, e_pad)
    s1p = segsum8(xhat, src, dst2d, z8)
    w1p = jnp.pad(W1, ((0, 8 - f), (0, 0)))
    h1hat = _make_tc2(n, pad)(s1p, xhat, dinv, w1p, b1.reshape(1, h))

    t8 = h1hat.reshape(n, 8, 8).transpose(1, 0, 2)
    s2c = _make_segsum8x8(n_pad, e_pad)(t8, src, dst2d, z8)
    s2 = s2c.transpose(1, 0, 2).reshape(n_pad, h)

    w3p = jnp.pad(W3, ((0, 0), (0, 7)))
    h3hatp = _make_tc3(n, pad)(s2, h1hat, dinv, W2, b2.reshape(1, h), w3p)

    s3p = segsum8(h3hatp, src, dst2d, z8)
    out = _make_tc4(n, pad)(s3p, h3hatp, dinv, b3.reshape(1, 1))
    return out


# speedup vs baseline: 26.1539x; 1.1427x over previous
import functools

import jax
import jax.numpy as jnp
from jax import lax
from jax.experimental import pallas as pl
from jax.experimental.pallas import tpu as pltpu
from jax.experimental.pallas import tpu_sc as plsc

NC = 2
NS = 16
ROW_BLK = 1280


def _mesh():
    return plsc.VectorSubcoreMesh(core_axis_name="c", subcore_axis_name="s")


_SC_PARAMS = pltpu.CompilerParams(use_tc_tiling_on_sc=False)


def _make_deg_count(n, e_pad):
    per_tile = e_pad // (NC * NS)
    k = 1024
    assert per_tile % k == 0 and n % (NS * ROW_BLK) == 0
    iters = per_tile // k
    rows_per = n // NS
    n_copies = rows_per // ROW_BLK

    @functools.partial(
        pl.kernel,
        out_type=jax.ShapeDtypeStruct((NC, n, 4), jnp.float32),
        mesh=_mesh(),
        compiler_params=_SC_PARAMS,
        scratch_types=[
            pltpu.VMEM((8, 128), jnp.int32),
            pltpu.VMEM((128, 4), jnp.float32),
            pltpu.VMEM((ROW_BLK, 4), jnp.float32),
            pltpu.VMEM_SHARED((n, 4), jnp.float32),
            pltpu.SemaphoreType.DMA,
        ],
    )
    def deg_kernel(dst_hbm, ones_hbm, zrows, out, dst_v, ones_v, zb, acc, sem):
        c = lax.axis_index("c")
        s = lax.axis_index("s")
        g = c * NS + s
        pltpu.sync_copy(ones_hbm, ones_v)
        pltpu.sync_copy(zrows, zb)
        rbase = s * rows_per
        for z in range(n_copies):
            pltpu.sync_copy(zb, acc.at[pl.ds(rbase + z * ROW_BLK, ROW_BLK)])
        plsc.subcore_barrier()
        row0 = g * (per_tile // 128)

        @pl.loop(0, iters)
        def _(i):
            pltpu.sync_copy(dst_hbm.at[pl.ds(row0 + i * (k // 128), k // 128)], dst_v)
            for j in range(k // 128):
                pltpu.sync_copy(ones_v, acc.at[dst_v.at[j]], add=True)

        plsc.subcore_barrier()
        for z in range(n_copies):
            sl = pl.ds(rbase + z * ROW_BLK, ROW_BLK)
            pltpu.sync_copy(acc.at[sl], out.at[c].at[sl])

    return deg_kernel


def _sweep_pipelined(table2d, src_hbm, dst_hbm, acc, srcb, dstb, rowsb, gsems,
                     ssem, ebase, row0, iters, k, m):
    def load(it, b):
        pltpu.sync_copy(src_hbm.at[pl.ds(ebase + it * k, k)], srcb[b])
        pltpu.sync_copy(dst_hbm.at[pl.ds(row0 + it * m, m)], dstb[b])

    def gather(b):
        return pltpu.async_copy(table2d.at[srcb[b]], rowsb[b], gsems[b])

    def scatter(b):
        ds_ = [
            pltpu.async_copy(rowsb[b].at[pl.ds(j * 128, 128)],
                             acc.at[dstb[b].at[j]], ssem, add=True)
            for j in range(m)
        ]
        for d in ds_:
            d.wait()

    load(0, 0)
    gather(0)

    @pl.loop(0, iters, step=2)
    def _(i):
        load(i + 1, 1)
        g1 = gather(1)
        pltpu.make_async_copy(table2d.at[srcb[0]], rowsb[0], gsems[0]).wait()
        scatter(0)

        @pl.when(i + 2 < iters)
        def _():
            load(i + 2, 0)
            gather(0)

        g1.wait()
        scatter(1)


def _make_segsum8(n, e_pad):
    per_tile = e_pad // (NC * NS)
    k = 1792
    m = k // 128
    assert per_tile % k == 0 and (per_tile // k) % 2 == 0
    iters = per_tile // k
    rows_per = n // NS
    n_copies = rows_per // ROW_BLK

    @functools.partial(
        pl.kernel,
        out_type=jax.ShapeDtypeStruct((NC, n, 8), jnp.float32),
        mesh=_mesh(),
        compiler_params=_SC_PARAMS,
        scratch_types=[
            pltpu.VMEM((k,), jnp.int32), pltpu.VMEM((k,), jnp.int32),
            pltpu.VMEM((m, 128), jnp.int32), pltpu.VMEM((m, 128), jnp.int32),
            pltpu.VMEM((k, 8), jnp.float32), pltpu.VMEM((k, 8), jnp.float32),
            pltpu.VMEM((ROW_BLK, 8), jnp.float32),
            pltpu.VMEM_SHARED((n, 8), jnp.float32),
            pltpu.SemaphoreType.DMA, pltpu.SemaphoreType.DMA,
            pltpu.SemaphoreType.DMA,
        ],
    )
    def segsum_kernel(table, src_hbm, dst_hbm, zrows, out,
                      s0, s1, d0, d1, r0, r1, zb, acc, gs0, gs1, ss):
        c = lax.axis_index("c")
        s = lax.axis_index("s")
        g = c * NS + s
        pltpu.sync_copy(zrows, zb)
        rbase = s * rows_per
        for z in range(n_copies):
            pltpu.sync_copy(zb, acc.at[pl.ds(rbase + z * ROW_BLK, ROW_BLK)])
        plsc.subcore_barrier()
        _sweep_pipelined(table, src_hbm, dst_hbm, acc, (s0, s1), (d0, d1),
                         (r0, r1), (gs0, gs1), ss, g * per_tile,
                         g * (per_tile // 128), iters, k, m)
        plsc.subcore_barrier()
        for z in range(n_copies):
            sl = pl.ds(rbase + z * ROW_BLK, ROW_BLK)
            pltpu.sync_copy(acc.at[sl], out.at[c].at[sl])

    return segsum_kernel


def _make_segsum16x4(n, e_pad):
    per_tile = e_pad // NS
    k = 512
    m = k // 128
    assert per_tile % k == 0 and (per_tile // k) % 2 == 0
    iters = per_tile // k
    rows_per = n // NS
    row_blk = 320
    n_copies = rows_per // row_blk

    @functools.partial(
        pl.kernel,
        out_type=jax.ShapeDtypeStruct((4, n, 16), jnp.float32),
        mesh=_mesh(),
        compiler_params=_SC_PARAMS,
        scratch_types=[
            pltpu.VMEM((k,), jnp.int32), pltpu.VMEM((k,), jnp.int32),
            pltpu.VMEM((m, 128), jnp.int32), pltpu.VMEM((m, 128), jnp.int32),
            pltpu.VMEM((k, 16), jnp.float32), pltpu.VMEM((k, 16), jnp.float32),
            pltpu.VMEM((row_blk, 16), jnp.float32),
            pltpu.VMEM_SHARED((n, 16), jnp.float32),
            pltpu.SemaphoreType.DMA, pltpu.SemaphoreType.DMA,
            pltpu.SemaphoreType.DMA,
        ],
    )
    def segsum_kernel(table, src_hbm, dst_hbm, zrows, out,
                      s0, s1, d0, d1, r0, r1, zb, acc, gs0, gs1, ss):
        c = lax.axis_index("c")
        s = lax.axis_index("s")
        pltpu.sync_copy(zrows, zb)
        rbase = s * rows_per
        ebase = s * per_tile
        row0 = s * (per_tile // 128)
        for cc in range(2):
            cid = c * 2 + cc
            for z in range(n_copies):
                pltpu.sync_copy(zb, acc.at[pl.ds(rbase + z * row_blk, row_blk)])
            plsc.subcore_barrier()
            _sweep_pipelined(table.at[cid], src_hbm, dst_hbm, acc, (s0, s1),
                             (d0, d1), (r0, r1), (gs0, gs1), ss, ebase, row0,
                             iters, k, m)
            plsc.subcore_barrier()
            for z in range(n_copies):
                sl = pl.ds(rbase + z * row_blk, row_blk)
                pltpu.sync_copy(acc.at[sl], out.at[cid].at[sl])
            plsc.subcore_barrier()

    return segsum_kernel


_R = 2000


def _row0_mask(pid, r):
    return (lax.broadcasted_iota(jnp.int32, (r, 1), 0) == 0) & (pid == 0)


def _make_tc1(n, pad_edges):
    def body(degp, xp, xhat, dinv):
        pid = pl.program_id(0)
        d = degp[...]
        deg = d[0, :, 0:1] + d[1, :, 0:1] + 1.0
        deg = deg - jnp.where(_row0_mask(pid, _R), float(pad_edges), 0.0)
        di = lax.rsqrt(deg)
        dinv[...] = di
        xhat[...] = xp[...] * di

    return pl.pallas_call(
        body,
        grid=(n // _R,),
        in_specs=[
            pl.BlockSpec((NC, _R, 4), lambda i: (0, i, 0)),
            pl.BlockSpec((_R, 8), lambda i: (i, 0)),
        ],
        out_specs=[
            pl.BlockSpec((_R, 8), lambda i: (i, 0)),
            pl.BlockSpec((_R, 1), lambda i: (i, 0)),
        ],
        out_shape=[
            jax.ShapeDtypeStruct((n, 8), jnp.float32),
            jax.ShapeDtypeStruct((n, 1), jnp.float32),
        ],
    )


def _make_tc2(n, pad_edges):
    def body(s1p, xhat, dinv, w1, b1, out):
        pid = pl.program_id(0)
        sp = s1p[...]
        xh = xhat[...]
        s = sp[0] + sp[1]
        s = s - jnp.where(_row0_mask(pid, _R), float(pad_edges) * xh[0:1, :], 0.0)
        a1 = dinv[...] * (s + xh)
        h1 = jnp.dot(a1, w1[...], preferred_element_type=jnp.float32,
                     precision=lax.Precision.HIGHEST) + b1[...]
        h1 = jnp.maximum(h1, 0.0) * dinv[...]
        out[...] = h1.reshape(_R, 4, 16).transpose(1, 0, 2)

    return pl.pallas_call(
        body,
        grid=(n // _R,),
        in_specs=[
            pl.BlockSpec((NC, _R, 8), lambda i: (0, i, 0)),
            pl.BlockSpec((_R, 8), lambda i: (i, 0)),
            pl.BlockSpec((_R, 1), lambda i: (i, 0)),
            pl.BlockSpec((8, 64), lambda i: (0, 0)),
            pl.BlockSpec((1, 64), lambda i: (0, 0)),
        ],
        out_specs=pl.BlockSpec((4, _R, 16), lambda i: (0, i, 0)),
        out_shape=jax.ShapeDtypeStruct((4, n, 16), jnp.float32),
    )


def _make_tc3(n, pad_edges):
    def body(s2c, t4, dinv, w2, b2, w3p, out):
        pid = pl.program_id(0)
        hh = t4[...].transpose(1, 0, 2).reshape(_R, 64)
        s = s2c[...].transpose(1, 0, 2).reshape(_R, 64)
        s = s - jnp.where(_row0_mask(pid, _R), float(pad_edges) * hh[0:1, :], 0.0)
        a2 = dinv[...] * (s + hh)
        h2 = jnp.dot(a2, w2[...], preferred_element_type=jnp.float32,
                     precision=lax.Precision.HIGHEST) + b2[...]
        h2 = jnp.maximum(h2, 0.0)
        t = jnp.dot(h2, w3p[...], preferred_element_type=jnp.float32,
                    precision=lax.Precision.HIGHEST)
        out[...] = t * dinv[...]

    return pl.pallas_call(
        body,
        grid=(n // _R,),
        in_specs=[
            pl.BlockSpec((4, _R, 16), lambda i: (0, i, 0)),
            pl.BlockSpec((4, _R, 16), lambda i: (0, i, 0)),
            pl.BlockSpec((_R, 1), lambda i: (i, 0)),
            pl.BlockSpec((64, 64), lambda i: (0, 0)),
            pl.BlockSpec((1, 64), lambda i: (0, 0)),
            pl.BlockSpec((64, 8), lambda i: (0, 0)),
        ],
        out_specs=pl.BlockSpec((_R, 8), lambda i: (i, 0)),
        out_shape=jax.ShapeDtypeStruct((n, 8), jnp.float32),
    )


def _make_tc4(n, pad_edges):
    def body(s3p, h3hat, dinv, b3, out):
        pid = pl.program_id(0)
        sp = s3p[...]
        hh = h3hat[...]
        s = sp[0] + sp[1]
        s = s - jnp.where(_row0_mask(pid, _R), float(pad_edges) * hh[0:1, :], 0.0)
        o = dinv[...] * (s + hh)
        out[...] = o[:, 0:1] + b3[...]

    return pl.pallas_call(
        body,
        grid=(n // _R,),
        in_specs=[
            pl.BlockSpec((NC, _R, 8), lambda i: (0, i, 0)),
            pl.BlockSpec((_R, 8), lambda i: (i, 0)),
            pl.BlockSpec((_R, 1), lambda i: (i, 0)),
            pl.BlockSpec((1, 1), lambda i: (0, 0)),
        ],
        out_specs=pl.BlockSpec((_R, 1), lambda i: (i, 0)),
        out_shape=jax.ShapeDtypeStruct((n, 1), jnp.float32),
    )


def kernel(x, edge_index, W1, b1, W2, b2, W3, b3):
    n, f = x.shape
    e = edge_index.shape[1]
    h = W1.shape[1]
    assert f == 5 and h == 64 and W3.shape[1] == 1

    unit = NC * NS * 1024
    e_pad = ((e + unit - 1) // unit) * unit
    pad = e_pad - e
    n_unit = NS * ROW_BLK
    n_pad = ((n + n_unit - 1) // n_unit) * n_unit

    src = jnp.concatenate([edge_index[0], jnp.zeros((pad,), jnp.int32)])
    dst = jnp.concatenate([edge_index[1], jnp.zeros((pad,), jnp.int32)])
    dst2d = dst.reshape(-1, 128)
    ones = jnp.ones((128, 4), jnp.float32)
    z4 = jnp.zeros((ROW_BLK, 4), jnp.float32)
    z8 = jnp.zeros((ROW_BLK, 8), jnp.float32)

    degp = _make_deg_count(n_pad, e_pad)(dst2d, ones, z4)
    xpad = jnp.pad(x, ((0, 0), (0, 8 - f)))
    xhat, dinv = _make_tc1(n, pad)(degp, xpad)

    segsum8 = _make_segsum8(n_pad, e_pad)
    s1p = segsum8(xhat, src, dst2d, z8)
    w1p = jnp.pad(W1, ((0, 8 - f), (0, 0)))
    h1hat = _make_tc2(n, pad)(s1p, xhat, dinv, w1p, b1.reshape(1, h))

    z16 = jnp.zeros((320, 16), jnp.float32)
    s2c = _make_segsum16x4(n_pad, e_pad)(h1hat, src, dst2d, z16)

    w3p = jnp.pad(W3, ((0, 0), (0, 7)))
    h3hatp = _make_tc3(n, pad)(s2c, h1hat, dinv, W2, b2.reshape(1, h), w3p)

    s3p = segsum8(h3hatp, src, dst2d, z8)
    out = _make_tc4(n, pad)(s3p, h3hatp, dinv, b3.reshape(1, 1))
    return out


# speedup vs baseline: 26.1554x; 1.0001x over previous
import functools

import jax
import jax.numpy as jnp
from jax import lax
from jax.experimental import pallas as pl
from jax.experimental.pallas import tpu as pltpu
from jax.experimental.pallas import tpu_sc as plsc

NC = 2
NS = 16
ROW_BLK = 1280


def _mesh():
    return plsc.VectorSubcoreMesh(core_axis_name="c", subcore_axis_name="s")


_SC_PARAMS = pltpu.CompilerParams(use_tc_tiling_on_sc=False)


def _make_deg_count(n, e_pad):
    per_tile = e_pad // (NC * NS)
    k = 1024
    assert per_tile % k == 0 and n % (NS * ROW_BLK) == 0
    iters = per_tile // k
    rows_per = n // NS
    n_copies = rows_per // ROW_BLK

    @functools.partial(
        pl.kernel,
        out_type=jax.ShapeDtypeStruct((NC, n, 4), jnp.float32),
        mesh=_mesh(),
        compiler_params=_SC_PARAMS,
        scratch_types=[
            pltpu.VMEM((8, 128), jnp.int32),
            pltpu.VMEM((128, 4), jnp.float32),
            pltpu.VMEM((ROW_BLK, 4), jnp.float32),
            pltpu.VMEM_SHARED((n, 4), jnp.float32),
            pltpu.SemaphoreType.DMA,
        ],
    )
    def deg_kernel(dst_hbm, ones_hbm, zrows, out, dst_v, ones_v, zb, acc, sem):
        c = lax.axis_index("c")
        s = lax.axis_index("s")
        g = c * NS + s
        pltpu.sync_copy(ones_hbm, ones_v)
        pltpu.sync_copy(zrows, zb)
        rbase = s * rows_per
        for z in range(n_copies):
            pltpu.sync_copy(zb, acc.at[pl.ds(rbase + z * ROW_BLK, ROW_BLK)])
        plsc.subcore_barrier()
        row0 = g * (per_tile // 128)

        @pl.loop(0, iters)
        def _(i):
            pltpu.sync_copy(dst_hbm.at[pl.ds(row0 + i * (k // 128), k // 128)], dst_v)
            for j in range(k // 128):
                pltpu.sync_copy(ones_v, acc.at[dst_v.at[j]], add=True)

        plsc.subcore_barrier()
        for z in range(n_copies):
            sl = pl.ds(rbase + z * ROW_BLK, ROW_BLK)
            pltpu.sync_copy(acc.at[sl], out.at[c].at[sl])

    return deg_kernel


def _sweep_pipelined(table2d, src_hbm, dst_hbm, acc, srcb, dstb, rowsb, gsems,
                     ssem, ebase, row0, iters, k, m):
    def load(it, b):
        pltpu.sync_copy(src_hbm.at[pl.ds(ebase + it * k, k)], srcb[b])
        pltpu.sync_copy(dst_hbm.at[pl.ds(ebase + it * k, k)], dstb[b])

    def gather(b):
        return pltpu.async_copy(table2d.at[srcb[b]], rowsb[b], gsems[b])

    def scatter(b):
        pltpu.async_copy(rowsb[b], acc.at[dstb[b]], ssem, add=True).wait()


    load(0, 0)
    gather(0)

    @pl.loop(0, iters, step=2)
    def _(i):
        load(i + 1, 1)
        g1 = gather(1)
        pltpu.make_async_copy(table2d.at[srcb[0]], rowsb[0], gsems[0]).wait()
        scatter(0)

        @pl.when(i + 2 < iters)
        def _():
            load(i + 2, 0)
            gather(0)

        g1.wait()
        scatter(1)


def _make_segsum8(n, e_pad):
    per_tile = e_pad // (NC * NS)
    k = 1792
    m = k // 128
    assert per_tile % k == 0 and (per_tile // k) % 2 == 0
    iters = per_tile // k
    rows_per = n // NS
    n_copies = rows_per // ROW_BLK

    @functools.partial(
        pl.kernel,
        out_type=jax.ShapeDtypeStruct((NC, n, 8), jnp.float32),
        mesh=_mesh(),
        compiler_params=_SC_PARAMS,
        scratch_types=[
            pltpu.VMEM((k,), jnp.int32), pltpu.VMEM((k,), jnp.int32),
            pltpu.VMEM((k,), jnp.int32), pltpu.VMEM((k,), jnp.int32),
            pltpu.VMEM((k, 8), jnp.float32), pltpu.VMEM((k, 8), jnp.float32),
            pltpu.VMEM((ROW_BLK, 8), jnp.float32),
            pltpu.VMEM_SHARED((n, 8), jnp.float32),
            pltpu.SemaphoreType.DMA, pltpu.SemaphoreType.DMA,
            pltpu.SemaphoreType.DMA,
        ],
    )
    def segsum_kernel(table, src_hbm, dst_hbm, zrows, out,
                      s0, s1, d0, d1, r0, r1, zb, acc, gs0, gs1, ss):
        c = lax.axis_index("c")
        s = lax.axis_index("s")
        g = c * NS + s
        pltpu.sync_copy(zrows, zb)
        rbase = s * rows_per
        for z in range(n_copies):
            pltpu.sync_copy(zb, acc.at[pl.ds(rbase + z * ROW_BLK, ROW_BLK)])
        plsc.subcore_barrier()
        _sweep_pipelined(table, src_hbm, dst_hbm, acc, (s0, s1), (d0, d1),
                         (r0, r1), (gs0, gs1), ss, g * per_tile,
                         g * (per_tile // 128), iters, k, m)
        plsc.subcore_barrier()
        for z in range(n_copies):
            sl = pl.ds(rbase + z * ROW_BLK, ROW_BLK)
            pltpu.sync_copy(acc.at[sl], out.at[c].at[sl])

    return segsum_kernel


def _make_segsum16x4(n, e_pad):
    per_tile = e_pad // NS
    k = 512
    m = k // 128
    assert per_tile % k == 0 and (per_tile // k) % 2 == 0
    iters = per_tile // k
    rows_per = n // NS
    row_blk = 320
    n_copies = rows_per // row_blk

    @functools.partial(
        pl.kernel,
        out_type=jax.ShapeDtypeStruct((4, n, 16), jnp.float32),
        mesh=_mesh(),
        compiler_params=_SC_PARAMS,
        scratch_types=[
            pltpu.VMEM((k,), jnp.int32), pltpu.VMEM((k,), jnp.int32),
            pltpu.VMEM((k,), jnp.int32), pltpu.VMEM((k,), jnp.int32),
            pltpu.VMEM((k, 16), jnp.float32), pltpu.VMEM((k, 16), jnp.float32),
            pltpu.VMEM((row_blk, 16), jnp.float32),
            pltpu.VMEM_SHARED((n, 16), jnp.float32),
            pltpu.SemaphoreType.DMA, pltpu.SemaphoreType.DMA,
            pltpu.SemaphoreType.DMA,
        ],
    )
    def segsum_kernel(table, src_hbm, dst_hbm, zrows, out,
                      s0, s1, d0, d1, r0, r1, zb, acc, gs0, gs1, ss):
        c = lax.axis_index("c")
        s = lax.axis_index("s")
        pltpu.sync_copy(zrows, zb)
        rbase = s * rows_per
        ebase = s * per_tile
        row0 = s * (per_tile // 128)
        for cc in range(2):
            cid = c * 2 + cc
            for z in range(n_copies):
                pltpu.sync_copy(zb, acc.at[pl.ds(rbase + z * row_blk, row_blk)])
            plsc.subcore_barrier()
            _sweep_pipelined(table.at[cid], src_hbm, dst_hbm, acc, (s0, s1),
                             (d0, d1), (r0, r1), (gs0, gs1), ss, ebase, row0,
                             iters, k, m)
            plsc.subcore_barrier()
            for z in range(n_copies):
                sl = pl.ds(rbase + z * row_blk, row_blk)
                pltpu.sync_copy(acc.at[sl], out.at[cid].at[sl])
            plsc.subcore_barrier()

    return segsum_kernel


_R = 2000


def _row0_mask(pid, r):
    return (lax.broadcasted_iota(jnp.int32, (r, 1), 0) == 0) & (pid == 0)


def _make_tc1(n, pad_edges):
    def body(degp, xp, xhat, dinv):
        pid = pl.program_id(0)
        d = degp[...]
        deg = d[0, :, 0:1] + d[1, :, 0:1] + 1.0
        deg = deg - jnp.where(_row0_mask(pid, _R), float(pad_edges), 0.0)
        di = lax.rsqrt(deg)
        dinv[...] = di
        xhat[...] = xp[...] * di

    return pl.pallas_call(
        body,
        grid=(n // _R,),
        in_specs=[
            pl.BlockSpec((NC, _R, 4), lambda i: (0, i, 0)),
            pl.BlockSpec((_R, 8), lambda i: (i, 0)),
        ],
        out_specs=[
            pl.BlockSpec((_R, 8), lambda i: (i, 0)),
            pl.BlockSpec((_R, 1), lambda i: (i, 0)),
        ],
        out_shape=[
            jax.ShapeDtypeStruct((n, 8), jnp.float32),
            jax.ShapeDtypeStruct((n, 1), jnp.float32),
        ],
    )


def _make_tc2(n, pad_edges):
    def body(s1p, xhat, dinv, w1, b1, out):
        pid = pl.program_id(0)
        sp = s1p[...]
        xh = xhat[...]
        s = sp[0] + sp[1]
        s = s - jnp.where(_row0_mask(pid, _R), float(pad_edges) * xh[0:1, :], 0.0)
        a1 = dinv[...] * (s + xh)
        h1 = jnp.dot(a1, w1[...], preferred_element_type=jnp.float32,
                     precision=lax.Precision.HIGHEST) + b1[...]
        h1 = jnp.maximum(h1, 0.0) * dinv[...]
        out[...] = h1.reshape(_R, 4, 16).transpose(1, 0, 2)

    return pl.pallas_call(
        body,
        grid=(n // _R,),
        in_specs=[
            pl.BlockSpec((NC, _R, 8), lambda i: (0, i, 0)),
            pl.BlockSpec((_R, 8), lambda i: (i, 0)),
            pl.BlockSpec((_R, 1), lambda i: (i, 0)),
            pl.BlockSpec((8, 64), lambda i: (0, 0)),
            pl.BlockSpec((1, 64), lambda i: (0, 0)),
        ],
        out_specs=pl.BlockSpec((4, _R, 16), lambda i: (0, i, 0)),
        out_shape=jax.ShapeDtypeStruct((4, n, 16), jnp.float32),
    )


def _make_tc3(n, pad_edges):
    def body(s2c, t4, dinv, w2, b2, w3p, out):
        pid = pl.program_id(0)
        hh = t4[...].transpose(1, 0, 2).reshape(_R, 64)
        s = s2c[...].transpose(1, 0, 2).reshape(_R, 64)
        s = s - jnp.where(_row0_mask(pid, _R), float(pad_edges) * hh[0:1, :], 0.0)
        a2 = dinv[...] * (s + hh)
        h2 = jnp.dot(a2, w2[...], preferred_element_type=jnp.float32,
                     precision=lax.Precision.HIGHEST) + b2[...]
        h2 = jnp.maximum(h2, 0.0)
        t = jnp.dot(h2, w3p[...], preferred_element_type=jnp.float32,
                    precision=lax.Precision.HIGHEST)
        out[...] = t * dinv[...]

    return pl.pallas_call(
        body,
        grid=(n // _R,),
        in_specs=[
            pl.BlockSpec((4, _R, 16), lambda i: (0, i, 0)),
            pl.BlockSpec((4, _R, 16), lambda i: (0, i, 0)),
            pl.BlockSpec((_R, 1), lambda i: (i, 0)),
            pl.BlockSpec((64, 64), lambda i: (0, 0)),
            pl.BlockSpec((1, 64), lambda i: (0, 0)),
            pl.BlockSpec((64, 8), lambda i: (0, 0)),
        ],
        out_specs=pl.BlockSpec((_R, 8), lambda i: (i, 0)),
        out_shape=jax.ShapeDtypeStruct((n, 8), jnp.float32),
    )


def _make_tc4(n, pad_edges):
    def body(s3p, h3hat, dinv, b3, out):
        pid = pl.program_id(0)
        sp = s3p[...]
        hh = h3hat[...]
        s = sp[0] + sp[1]
        s = s - jnp.where(_row0_mask(pid, _R), float(pad_edges) * hh[0:1, :], 0.0)
        o = dinv[...] * (s + hh)
        out[...] = o[:, 0:1] + b3[...]

    return pl.pallas_call(
        body,
        grid=(n // _R,),
        in_specs=[
            pl.BlockSpec((NC, _R, 8), lambda i: (0, i, 0)),
            pl.BlockSpec((_R, 8), lambda i: (i, 0)),
            pl.BlockSpec((_R, 1), lambda i: (i, 0)),
            pl.BlockSpec((1, 1), lambda i: (0, 0)),
        ],
        out_specs=pl.BlockSpec((_R, 1), lambda i: (i, 0)),
        out_shape=jax.ShapeDtypeStruct((n, 1), jnp.float32),
    )


def kernel(x, edge_index, W1, b1, W2, b2, W3, b3):
    n, f = x.shape
    e = edge_index.shape[1]
    h = W1.shape[1]
    assert f == 5 and h == 64 and W3.shape[1] == 1

    unit = NC * NS * 1024
    e_pad = ((e + unit - 1) // unit) * unit
    pad = e_pad - e
    n_unit = NS * ROW_BLK
    n_pad = ((n + n_unit - 1) // n_unit) * n_unit

    src = jnp.concatenate([edge_index[0], jnp.zeros((pad,), jnp.int32)])
    dst = jnp.concatenate([edge_index[1], jnp.zeros((pad,), jnp.int32)])
    dst2d = dst.reshape(-1, 128)
    ones = jnp.ones((128, 4), jnp.float32)
    z4 = jnp.zeros((ROW_BLK, 4), jnp.float32)
    z8 = jnp.zeros((ROW_BLK, 8), jnp.float32)

    degp = _make_deg_count(n_pad, e_pad)(dst2d, ones, z4)
    xpad = jnp.pad(x, ((0, 0), (0, 8 - f)))
    xhat, dinv = _make_tc1(n, pad)(degp, xpad)

    segsum8 = _make_segsum8(n_pad, e_pad)
    s1p = segsum8(xhat, src, dst, z8)
    w1p = jnp.pad(W1, ((0, 8 - f), (0, 0)))
    h1hat = _make_tc2(n, pad)(s1p, xhat, dinv, w1p, b1.reshape(1, h))

    z16 = jnp.zeros((320, 16), jnp.float32)
    s2c = _make_segsum16x4(n_pad, e_pad)(h1hat, src, dst, z16)

    w3p = jnp.pad(W3, ((0, 0), (0, 7)))
    h3hatp = _make_tc3(n, pad)(s2c, h1hat, dinv, W2, b2.reshape(1, h), w3p)

    s3p = segsum8(h3hatp, src, dst, z8)
    out = _make_tc4(n, pad)(s3p, h3hatp, dinv, b3.reshape(1, 1))
    return out


# speedup vs baseline: 29.4831x; 1.1272x over previous
import functools

import jax
import jax.numpy as jnp
from jax import lax
from jax.experimental import pallas as pl
from jax.experimental.pallas import tpu as pltpu
from jax.experimental.pallas import tpu_sc as plsc

NC = 2
NS = 16
ROW_BLK = 1280


def _mesh():
    return plsc.VectorSubcoreMesh(core_axis_name="c", subcore_axis_name="s")


_SC_PARAMS = pltpu.CompilerParams(use_tc_tiling_on_sc=False)


def _make_deg_count(n, e_pad):
    per_tile = e_pad // (NC * NS)
    k = 1024
    assert per_tile % k == 0 and n % (NS * ROW_BLK) == 0
    iters = per_tile // k
    rows_per = n // NS
    n_copies = rows_per // ROW_BLK

    @functools.partial(
        pl.kernel,
        out_type=jax.ShapeDtypeStruct((NC, n, 4), jnp.float32),
        mesh=_mesh(),
        compiler_params=_SC_PARAMS,
        scratch_types=[
            pltpu.VMEM((8, 128), jnp.int32),
            pltpu.VMEM((128, 4), jnp.float32),
            pltpu.VMEM((ROW_BLK, 4), jnp.float32),
            pltpu.VMEM_SHARED((n, 4), jnp.float32),
            pltpu.SemaphoreType.DMA,
        ],
    )
    def deg_kernel(dst_hbm, ones_hbm, zrows, out, dst_v, ones_v, zb, acc, sem):
        c = lax.axis_index("c")
        s = lax.axis_index("s")
        g = c * NS + s
        pltpu.sync_copy(ones_hbm, ones_v)
        pltpu.sync_copy(zrows, zb)
        rbase = s * rows_per
        for z in range(n_copies):
            pltpu.sync_copy(zb, acc.at[pl.ds(rbase + z * ROW_BLK, ROW_BLK)])
        plsc.subcore_barrier()
        row0 = g * (per_tile // 128)

        @pl.loop(0, iters)
        def _(i):
            pltpu.sync_copy(dst_hbm.at[pl.ds(row0 + i * (k // 128), k // 128)], dst_v)
            for j in range(k // 128):
                pltpu.sync_copy(ones_v, acc.at[dst_v.at[j]], add=True)

        plsc.subcore_barrier()
        for z in range(n_copies):
            sl = pl.ds(rbase + z * ROW_BLK, ROW_BLK)
            pltpu.sync_copy(acc.at[sl], out.at[c].at[sl])

    return deg_kernel


def _sweep_pipelined(table2d, src_hbm, dst_hbm, acc, srcb, dstb, rowsb, gsems,
                     ssem, sbase, dbase, iters, k, m):
    def load(it, b):
        pltpu.sync_copy(src_hbm.at[pl.ds(sbase + it * k, k)], srcb[b])
        pltpu.sync_copy(dst_hbm.at[pl.ds(dbase + it * k, k)], dstb[b])

    def gather(b):
        return pltpu.async_copy(table2d.at[srcb[b]], rowsb[b], gsems[b])

    def scatter(b):
        pltpu.async_copy(rowsb[b], acc.at[dstb[b]], ssem, add=True).wait()


    load(0, 0)
    gather(0)

    @pl.loop(0, iters, step=2)
    def _(i):
        load(i + 1, 1)
        g1 = gather(1)
        pltpu.make_async_copy(table2d.at[srcb[0]], rowsb[0], gsems[0]).wait()
        scatter(0)

        @pl.when(i + 2 < iters)
        def _():
            load(i + 2, 0)
            gather(0)

        g1.wait()
        scatter(1)


def _make_segsum8(n, e_pad):
    per_tile = e_pad // (NC * NS)
    k = 1792
    m = k // 128
    assert per_tile % k == 0 and (per_tile // k) % 2 == 0
    iters = per_tile // k
    rows_per = n // NS
    n_copies = rows_per // ROW_BLK

    @functools.partial(
        pl.kernel,
        out_type=jax.ShapeDtypeStruct((NC, n, 8), jnp.float32),
        mesh=_mesh(),
        compiler_params=_SC_PARAMS,
        scratch_types=[
            pltpu.VMEM((k,), jnp.int32), pltpu.VMEM((k,), jnp.int32),
            pltpu.VMEM((k,), jnp.int32), pltpu.VMEM((k,), jnp.int32),
            pltpu.VMEM((k, 8), jnp.float32), pltpu.VMEM((k, 8), jnp.float32),
            pltpu.VMEM((ROW_BLK, 8), jnp.float32),
            pltpu.VMEM_SHARED((n, 8), jnp.float32),
            pltpu.SemaphoreType.DMA, pltpu.SemaphoreType.DMA,
            pltpu.SemaphoreType.DMA,
        ],
    )
    def segsum_kernel(table, src_hbm, dst_hbm, zrows, out,
                      s0, s1, d0, d1, r0, r1, zb, acc, gs0, gs1, ss):
        c = lax.axis_index("c")
        s = lax.axis_index("s")
        g = c * NS + s
        pltpu.sync_copy(zrows, zb)
        rbase = s * rows_per
        for z in range(n_copies):
            pltpu.sync_copy(zb, acc.at[pl.ds(rbase + z * ROW_BLK, ROW_BLK)])
        plsc.subcore_barrier()
        _sweep_pipelined(table, src_hbm, dst_hbm, acc, (s0, s1), (d0, d1),
                         (r0, r1), (gs0, gs1), ss, g * per_tile,
                         g * per_tile, iters, k, m)
        plsc.subcore_barrier()
        for z in range(n_copies):
            sl = pl.ds(rbase + z * ROW_BLK, ROW_BLK)
            pltpu.sync_copy(acc.at[sl], out.at[c].at[sl])

    return segsum_kernel


def _make_segsum16x4(n, e_pad):
    per_tile = e_pad // NS
    k = 512
    m = k // 128
    assert per_tile % k == 0 and (per_tile // k) % 2 == 0
    iters = per_tile // k
    rows_per = n // NS
    row_blk = 320
    n_copies = rows_per // row_blk

    @functools.partial(
        pl.kernel,
        out_type=jax.ShapeDtypeStruct((n, 64), jnp.float32),
        mesh=_mesh(),
        compiler_params=_SC_PARAMS,
        scratch_types=[
            pltpu.VMEM((k,), jnp.int32), pltpu.VMEM((k,), jnp.int32),
            pltpu.VMEM((k,), jnp.int32), pltpu.VMEM((k,), jnp.int32),
            pltpu.VMEM((k, 16), jnp.float32), pltpu.VMEM((k, 16), jnp.float32),
            pltpu.VMEM((row_blk, 16), jnp.float32),
            pltpu.VMEM_SHARED((n, 16), jnp.float32),
            pltpu.SemaphoreType.DMA, pltpu.SemaphoreType.DMA,
            pltpu.SemaphoreType.DMA,
        ],
    )
    def segsum_kernel(table, src_hbm, dst_hbm, zrows, out,
                      s0, s1, d0, d1, r0, r1, zb, acc, gs0, gs1, ss):
        c = lax.axis_index("c")
        s = lax.axis_index("s")
        pltpu.sync_copy(zrows, zb)
        rbase = s * rows_per
        ebase = s * per_tile
        for cc in range(2):
            cid = c * 2 + cc
            for z in range(n_copies):
                pltpu.sync_copy(zb, acc.at[pl.ds(rbase + z * row_blk, row_blk)])
            plsc.subcore_barrier()
            _sweep_pipelined(table, src_hbm, dst_hbm, acc, (s0, s1),
                             (d0, d1), (r0, r1), (gs0, gs1), ss,
                             cid * e_pad + ebase, ebase, iters, k, m)
            plsc.subcore_barrier()
            for z in range(n_copies):
                sl = pl.ds(rbase + z * row_blk, row_blk)
                pltpu.sync_copy(acc.at[sl], out.at[sl, pl.ds(cid * 16, 16)])
            plsc.subcore_barrier()

    return segsum_kernel


_R = 2000


def _row0_mask(pid, r):
    return (lax.broadcasted_iota(jnp.int32, (r, 1), 0) == 0) & (pid == 0)


def _make_tc1(n, pad_edges):
    def body(degp, xp, xhat, dinv):
        pid = pl.program_id(0)
        d = degp[...]
        deg = d[0, :, 0:1] + d[1, :, 0:1] + 1.0
        deg = deg - jnp.where(_row0_mask(pid, _R), float(pad_edges), 0.0)
        di = lax.rsqrt(deg)
        dinv[...] = di
        xhat[...] = xp[...] * di

    return pl.pallas_call(
        body,
        grid=(n // _R,),
        in_specs=[
            pl.BlockSpec((NC, _R, 4), lambda i: (0, i, 0)),
            pl.BlockSpec((_R, 8), lambda i: (i, 0)),
        ],
        out_specs=[
            pl.BlockSpec((_R, 8), lambda i: (i, 0)),
            pl.BlockSpec((_R, 1), lambda i: (i, 0)),
        ],
        out_shape=[
            jax.ShapeDtypeStruct((n, 8), jnp.float32),
            jax.ShapeDtypeStruct((n, 1), jnp.float32),
        ],
    )


def _make_tc2(n, pad_edges):
    def body(s1p, xhat, dinv, w1, b1, out):
        pid = pl.program_id(0)
        sp = s1p[...]
        xh = xhat[...]
        s = sp[0] + sp[1]
        s = s - jnp.where(_row0_mask(pid, _R), float(pad_edges) * xh[0:1, :], 0.0)
        a1 = dinv[...] * (s + xh)
        h1 = jnp.dot(a1, w1[...], preferred_element_type=jnp.float32,
                     precision=lax.Precision.HIGHEST) + b1[...]
        out[...] = jnp.maximum(h1, 0.0) * dinv[...]

    return pl.pallas_call(
        body,
        grid=(n // _R,),
        in_specs=[
            pl.BlockSpec((NC, _R, 8), lambda i: (0, i, 0)),
            pl.BlockSpec((_R, 8), lambda i: (i, 0)),
            pl.BlockSpec((_R, 1), lambda i: (i, 0)),
            pl.BlockSpec((8, 64), lambda i: (0, 0)),
            pl.BlockSpec((1, 64), lambda i: (0, 0)),
        ],
        out_specs=pl.BlockSpec((_R, 64), lambda i: (i, 0)),
        out_shape=jax.ShapeDtypeStruct((n, 64), jnp.float32),
    )


def _make_tc3(n, pad_edges):
    def body(s2c, t4, dinv, w2, b2, w3p, out):
        pid = pl.program_id(0)
        hh = t4[...]
        s = s2c[...]
        s = s - jnp.where(_row0_mask(pid, _R), float(pad_edges) * hh[0:1, :], 0.0)
        a2 = dinv[...] * (s + hh)
        h2 = jnp.dot(a2, w2[...], preferred_element_type=jnp.float32,
                     precision=lax.Precision.HIGHEST) + b2[...]
        h2 = jnp.maximum(h2, 0.0)
        t = jnp.dot(h2, w3p[...], preferred_element_type=jnp.float32,
                    precision=lax.Precision.HIGHEST)
        out[...] = t * dinv[...]

    return pl.pallas_call(
        body,
        grid=(n // _R,),
        in_specs=[
            pl.BlockSpec((_R, 64), lambda i: (i, 0)),
            pl.BlockSpec((_R, 64), lambda i: (i, 0)),
            pl.BlockSpec((_R, 1), lambda i: (i, 0)),
            pl.BlockSpec((64, 64), lambda i: (0, 0)),
            pl.BlockSpec((1, 64), lambda i: (0, 0)),
            pl.BlockSpec((64, 8), lambda i: (0, 0)),
        ],
        out_specs=pl.BlockSpec((_R, 8), lambda i: (i, 0)),
        out_shape=jax.ShapeDtypeStruct((n, 8), jnp.float32),
    )


def _make_tc4(n, pad_edges):
    def body(s3p, h3hat, dinv, b3, out):
        pid = pl.program_id(0)
        sp = s3p[...]
        hh = h3hat[...]
        s = sp[0] + sp[1]
        s = s - jnp.where(_row0_mask(pid, _R), float(pad_edges) * hh[0:1, :], 0.0)
        o = dinv[...] * (s + hh)
        out[...] = o[:, 0:1] + b3[...]

    return pl.pallas_call(
        body,
        grid=(n // _R,),
        in_specs=[
            pl.BlockSpec((NC, _R, 8), lambda i: (0, i, 0)),
            pl.BlockSpec((_R, 8), lambda i: (i, 0)),
            pl.BlockSpec((_R, 1), lambda i: (i, 0)),
            pl.BlockSpec((1, 1), lambda i: (0, 0)),
        ],
        out_specs=pl.BlockSpec((_R, 1), lambda i: (i, 0)),
        out_shape=jax.ShapeDtypeStruct((n, 1), jnp.float32),
    )


def kernel(x, edge_index, W1, b1, W2, b2, W3, b3):
    n, f = x.shape
    e = edge_index.shape[1]
    h = W1.shape[1]
    assert f == 5 and h == 64 and W3.shape[1] == 1

    unit = NC * NS * 1024
    e_pad = ((e + unit - 1) // unit) * unit
    pad = e_pad - e
    n_unit = NS * ROW_BLK
    n_pad = ((n + n_unit - 1) // n_unit) * n_unit

    src = jnp.concatenate([edge_index[0], jnp.zeros((pad,), jnp.int32)])
    dst = jnp.concatenate([edge_index[1], jnp.zeros((pad,), jnp.int32)])
    dst2d = dst.reshape(-1, 128)
    ones = jnp.ones((128, 4), jnp.float32)
    z4 = jnp.zeros((ROW_BLK, 4), jnp.float32)
    z8 = jnp.zeros((ROW_BLK, 8), jnp.float32)

    degp = _make_deg_count(n_pad, e_pad)(dst2d, ones, z4)
    xpad = jnp.pad(x, ((0, 0), (0, 8 - f)))
    xhat, dinv = _make_tc1(n, pad)(degp, xpad)

    segsum8 = _make_segsum8(n_pad, e_pad)
    s1p = segsum8(xhat, src, dst, z8)
    w1p = jnp.pad(W1, ((0, 8 - f), (0, 0)))
    h1hat = _make_tc2(n, pad)(s1p, xhat, dinv, w1p, b1.reshape(1, h))

    z16 = jnp.zeros((320, 16), jnp.float32)
    h1lin = h1hat.reshape(4 * n, 16)
    src4 = src * 4
    src4cat = jnp.concatenate([src4, src4 + 1, src4 + 2, src4 + 3])
    s2m = _make_segsum16x4(n_pad, e_pad)(h1lin, src4cat, dst, z16)

    w3p = jnp.pad(W3, ((0, 0), (0, 7)))
    h3hatp = _make_tc3(n, pad)(s2m, h1hat, dinv, W2, b2.reshape(1, h), w3p)

    s3p = segsum8(h3hatp, src, dst, z8)
    out = _make_tc4(n, pad)(s3p, h3hatp, dinv, b3.reshape(1, 1))
    return out


# speedup vs baseline: 32.4765x; 1.1015x over previous
import functools

import jax
import jax.numpy as jnp
from jax import lax
from jax.experimental import pallas as pl
from jax.experimental.pallas import tpu as pltpu
from jax.experimental.pallas import tpu_sc as plsc

NC = 2
NS = 16
ROW_BLK = 1280


def _mesh():
    return plsc.VectorSubcoreMesh(core_axis_name="c", subcore_axis_name="s")


_SC_PARAMS = pltpu.CompilerParams(use_tc_tiling_on_sc=False)


def _make_deg_count(n, e_pad):
    per_tile = e_pad // (NC * NS)
    k = 1024
    assert per_tile % k == 0 and n % (NS * ROW_BLK) == 0
    iters = per_tile // k
    rows_per = n // NS
    n_copies = rows_per // ROW_BLK

    @functools.partial(
        pl.kernel,
        out_type=jax.ShapeDtypeStruct((NC, n, 4), jnp.float32),
        mesh=_mesh(),
        compiler_params=_SC_PARAMS,
        scratch_types=[
            pltpu.VMEM((8, 128), jnp.int32),
            pltpu.VMEM((128, 4), jnp.float32),
            pltpu.VMEM((ROW_BLK, 4), jnp.float32),
            pltpu.VMEM_SHARED((n, 4), jnp.float32),
            pltpu.SemaphoreType.DMA,
        ],
    )
    def deg_kernel(dst_hbm, ones_hbm, zrows, out, dst_v, ones_v, zb, acc, sem):
        c = lax.axis_index("c")
        s = lax.axis_index("s")
        g = c * NS + s
        pltpu.sync_copy(ones_hbm, ones_v)
        pltpu.sync_copy(zrows, zb)
        rbase = s * rows_per
        for z in range(n_copies):
            pltpu.sync_copy(zb, acc.at[pl.ds(rbase + z * ROW_BLK, ROW_BLK)])
        plsc.subcore_barrier()
        row0 = g * (per_tile // 128)

        @pl.loop(0, iters)
        def _(i):
            pltpu.sync_copy(dst_hbm.at[pl.ds(row0 + i * (k // 128), k // 128)], dst_v)
            for j in range(k // 128):
                pltpu.sync_copy(ones_v, acc.at[dst_v.at[j]], add=True)

        plsc.subcore_barrier()
        for z in range(n_copies):
            sl = pl.ds(rbase + z * ROW_BLK, ROW_BLK)
            pltpu.sync_copy(acc.at[sl], out.at[c].at[sl])

    return deg_kernel


def _sweep_pipelined(table2d, src_hbm, dst_hbm, acc, srcb, dstb, rowsb, gsems,
                     ssem, isem, sbase, dbase, iters, k, m):
    def load(it, b):
        pltpu.async_copy(src_hbm.at[pl.ds(sbase + it * k, k)], srcb[b], isem)
        pltpu.async_copy(dst_hbm.at[pl.ds(dbase + it * k, k)], dstb[b], isem)

    def wait_idx(b):
        pltpu.make_async_copy(src_hbm.at[pl.ds(sbase, k)], srcb[b], isem).wait()
        pltpu.make_async_copy(dst_hbm.at[pl.ds(dbase, k)], dstb[b], isem).wait()

    def gather(b):
        pltpu.async_copy(table2d.at[srcb[b]], rowsb[b], gsems[b])

    def wait_gather(b):
        pltpu.make_async_copy(table2d.at[srcb[b]], rowsb[b], gsems[b]).wait()

    def scatter(b):
        pltpu.async_copy(rowsb[b], acc.at[dstb[b]], ssem, add=True).wait()

    load(0, 0)
    wait_idx(0)
    gather(0)
    load(1, 1)

    @pl.loop(0, iters, step=2)
    def _(i):
        wait_idx(1)
        gather(1)
        wait_gather(0)

        @pl.when(i + 2 < iters)
        def _():
            load(i + 2, 0)

        scatter(0)

        @pl.when(i + 2 < iters)
        def _():
            wait_idx(0)
            gather(0)

        wait_gather(1)

        @pl.when(i + 3 < iters)
        def _():
            load(i + 3, 1)

        scatter(1)


def _make_segsum8(n, e_pad):
    per_tile = e_pad // (NC * NS)
    k = 1792
    m = k // 128
    assert per_tile % k == 0 and (per_tile // k) % 2 == 0
    iters = per_tile // k
    rows_per = n // NS
    n_copies = rows_per // ROW_BLK

    @functools.partial(
        pl.kernel,
        out_type=jax.ShapeDtypeStruct((NC, n, 8), jnp.float32),
        mesh=_mesh(),
        compiler_params=_SC_PARAMS,
        scratch_types=[
            pltpu.VMEM((k,), jnp.int32), pltpu.VMEM((k,), jnp.int32),
            pltpu.VMEM((k,), jnp.int32), pltpu.VMEM((k,), jnp.int32),
            pltpu.VMEM((k, 8), jnp.float32), pltpu.VMEM((k, 8), jnp.float32),
            pltpu.VMEM((ROW_BLK, 8), jnp.float32),
            pltpu.VMEM_SHARED((n, 8), jnp.float32),
            pltpu.SemaphoreType.DMA, pltpu.SemaphoreType.DMA,
            pltpu.SemaphoreType.DMA, pltpu.SemaphoreType.DMA,
        ],
    )
    def segsum_kernel(table, src_hbm, dst_hbm, zrows, out,
                      s0, s1, d0, d1, r0, r1, zb, acc, gs0, gs1, ss, isem):
        c = lax.axis_index("c")
        s = lax.axis_index("s")
        g = c * NS + s
        pltpu.sync_copy(zrows, zb)
        rbase = s * rows_per
        for z in range(n_copies):
            pltpu.sync_copy(zb, acc.at[pl.ds(rbase + z * ROW_BLK, ROW_BLK)])
        plsc.subcore_barrier()
        _sweep_pipelined(table, src_hbm, dst_hbm, acc, (s0, s1), (d0, d1),
                         (r0, r1), (gs0, gs1), ss, isem, g * per_tile,
                         g * per_tile, iters, k, m)
        plsc.subcore_barrier()
        for z in range(n_copies):
            sl = pl.ds(rbase + z * ROW_BLK, ROW_BLK)
            pltpu.sync_copy(acc.at[sl], out.at[c].at[sl])

    return segsum_kernel


def _make_segsum16x4(n, e_pad):
    per_tile = e_pad // NS
    k = 512
    m = k // 128
    assert per_tile % k == 0 and (per_tile // k) % 2 == 0
    iters = per_tile // k
    rows_per = n // NS
    row_blk = 320
    n_copies = rows_per // row_blk

    @functools.partial(
        pl.kernel,
        out_type=jax.ShapeDtypeStruct((n, 64), jnp.float32),
        mesh=_mesh(),
        compiler_params=_SC_PARAMS,
        scratch_types=[
            pltpu.VMEM((k,), jnp.int32), pltpu.VMEM((k,), jnp.int32),
            pltpu.VMEM((k,), jnp.int32), pltpu.VMEM((k,), jnp.int32),
            pltpu.VMEM((k, 16), jnp.float32), pltpu.VMEM((k, 16), jnp.float32),
            pltpu.VMEM((row_blk, 16), jnp.float32),
            pltpu.VMEM_SHARED((n, 16), jnp.float32),
            pltpu.SemaphoreType.DMA, pltpu.SemaphoreType.DMA,
            pltpu.SemaphoreType.DMA, pltpu.SemaphoreType.DMA,
        ],
    )
    def segsum_kernel(table, src_hbm, dst_hbm, zrows, out,
                      s0, s1, d0, d1, r0, r1, zb, acc, gs0, gs1, ss, isem):
        c = lax.axis_index("c")
        s = lax.axis_index("s")
        pltpu.sync_copy(zrows, zb)
        rbase = s * rows_per
        ebase = s * per_tile
        for cc in range(2):
            cid = c * 2 + cc
            for z in range(n_copies):
                pltpu.sync_copy(zb, acc.at[pl.ds(rbase + z * row_blk, row_blk)])
            plsc.subcore_barrier()
            _sweep_pipelined(table, src_hbm, dst_hbm, acc, (s0, s1),
                             (d0, d1), (r0, r1), (gs0, gs1), ss, isem,
                             cid * e_pad + ebase, ebase, iters, k, m)
            plsc.subcore_barrier()
            for z in range(n_copies):
                sl = pl.ds(rbase + z * row_blk, row_blk)
                pltpu.sync_copy(acc.at[sl], out.at[sl, pl.ds(cid * 16, 16)])
            plsc.subcore_barrier()

    return segsum_kernel


_R = 2000


def _row0_mask(pid, r):
    return (lax.broadcasted_iota(jnp.int32, (r, 1), 0) == 0) & (pid == 0)


def _make_tc1(n, pad_edges):
    def body(degp, xp, xhat, dinv):
        pid = pl.program_id(0)
        d = degp[...]
        deg = d[0, :, 0:1] + d[1, :, 0:1] + 1.0
        deg = deg - jnp.where(_row0_mask(pid, _R), float(pad_edges), 0.0)
        di = lax.rsqrt(deg)
        dinv[...] = di
        xhat[...] = xp[...] * di

    return pl.pallas_call(
        body,
        grid=(n // _R,),
        in_specs=[
            pl.BlockSpec((NC, _R, 4), lambda i: (0, i, 0)),
            pl.BlockSpec((_R, 8), lambda i: (i, 0)),
        ],
        out_specs=[
            pl.BlockSpec((_R, 8), lambda i: (i, 0)),
            pl.BlockSpec((_R, 1), lambda i: (i, 0)),
        ],
        out_shape=[
            jax.ShapeDtypeStruct((n, 8), jnp.float32),
            jax.ShapeDtypeStruct((n, 1), jnp.float32),
        ],
    )


def _make_tc2(n, pad_edges):
    def body(s1p, xhat, dinv, w1, b1, out):
        pid = pl.program_id(0)
        sp = s1p[...]
        xh = xhat[...]
        s = sp[0] + sp[1]
        s = s - jnp.where(_row0_mask(pid, _R), float(pad_edges) * xh[0:1, :], 0.0)
        a1 = dinv[...] * (s + xh)
        h1 = jnp.dot(a1, w1[...], preferred_element_type=jnp.float32,
                     precision=lax.Precision.HIGHEST) + b1[...]
        out[...] = jnp.maximum(h1, 0.0) * dinv[...]

    return pl.pallas_call(
        body,
        grid=(n // _R,),
        in_specs=[
            pl.BlockSpec((NC, _R, 8), lambda i: (0, i, 0)),
            pl.BlockSpec((_R, 8), lambda i: (i, 0)),
            pl.BlockSpec((_R, 1), lambda i: (i, 0)),
            pl.BlockSpec((8, 64), lambda i: (0, 0)),
            pl.BlockSpec((1, 64), lambda i: (0, 0)),
        ],
        out_specs=pl.BlockSpec((_R, 64), lambda i: (i, 0)),
        out_shape=jax.ShapeDtypeStruct((n, 64), jnp.float32),
    )


def _make_tc3(n, pad_edges):
    def body(s2c, t4, dinv, w2, b2, w3p, out):
        pid = pl.program_id(0)
        hh = t4[...]
        s = s2c[...]
        s = s - jnp.where(_row0_mask(pid, _R), float(pad_edges) * hh[0:1, :], 0.0)
        a2 = dinv[...] * (s + hh)
        h2 = jnp.dot(a2, w2[...], preferred_element_type=jnp.float32,
                     precision=lax.Precision.HIGHEST) + b2[...]
        h2 = jnp.maximum(h2, 0.0)
        t = jnp.dot(h2, w3p[...], preferred_element_type=jnp.float32,
                    precision=lax.Precision.HIGHEST)
        out[...] = t * dinv[...]

    return pl.pallas_call(
        body,
        grid=(n // _R,),
        in_specs=[
            pl.BlockSpec((_R, 64), lambda i: (i, 0)),
            pl.BlockSpec((_R, 64), lambda i: (i, 0)),
            pl.BlockSpec((_R, 1), lambda i: (i, 0)),
            pl.BlockSpec((64, 64), lambda i: (0, 0)),
            pl.BlockSpec((1, 64), lambda i: (0, 0)),
            pl.BlockSpec((64, 8), lambda i: (0, 0)),
        ],
        out_specs=pl.BlockSpec((_R, 8), lambda i: (i, 0)),
        out_shape=jax.ShapeDtypeStruct((n, 8), jnp.float32),
    )


def _make_tc4(n, pad_edges):
    def body(s3p, h3hat, dinv, b3, out):
        pid = pl.program_id(0)
        sp = s3p[...]
        hh = h3hat[...]
        s = sp[0] + sp[1]
        s = s - jnp.where(_row0_mask(pid, _R), float(pad_edges) * hh[0:1, :], 0.0)
        o = dinv[...] * (s + hh)
        out[...] = o[:, 0:1] + b3[...]

    return pl.pallas_call(
        body,
        grid=(n // _R,),
        in_specs=[
            pl.BlockSpec((NC, _R, 8), lambda i: (0, i, 0)),
            pl.BlockSpec((_R, 8), lambda i: (i, 0)),
            pl.BlockSpec((_R, 1), lambda i: (i, 0)),
            pl.BlockSpec((1, 1), lambda i: (0, 0)),
        ],
        out_specs=pl.BlockSpec((_R, 1), lambda i: (i, 0)),
        out_shape=jax.ShapeDtypeStruct((n, 1), jnp.float32),
    )


def kernel(x, edge_index, W1, b1, W2, b2, W3, b3):
    n, f = x.shape
    e = edge_index.shape[1]
    h = W1.shape[1]
    assert f == 5 and h == 64 and W3.shape[1] == 1

    unit = NC * NS * 1024
    e_pad = ((e + unit - 1) // unit) * unit
    pad = e_pad - e
    n_unit = NS * ROW_BLK
    n_pad = ((n + n_unit - 1) // n_unit) * n_unit

    src = jnp.concatenate([edge_index[0], jnp.zeros((pad,), jnp.int32)])
    dst = jnp.concatenate([edge_index[1], jnp.zeros((pad,), jnp.int32)])
    dst2d = dst.reshape(-1, 128)
    ones = jnp.ones((128, 4), jnp.float32)
    z4 = jnp.zeros((ROW_BLK, 4), jnp.float32)
    z8 = jnp.zeros((ROW_BLK, 8), jnp.float32)

    degp = _make_deg_count(n_pad, e_pad)(dst2d, ones, z4)
    xpad = jnp.pad(x, ((0, 0), (0, 8 - f)))
    xhat, dinv = _make_tc1(n, pad)(degp, xpad)

    segsum8 = _make_segsum8(n_pad, e_pad)
    s1p = segsum8(xhat, src, dst, z8)
    w1p = jnp.pad(W1, ((0, 8 - f), (0, 0)))
    h1hat = _make_tc2(n, pad)(s1p, xhat, dinv, w1p, b1.reshape(1, h))

    z16 = jnp.zeros((320, 16), jnp.float32)
    h1lin = h1hat.reshape(4 * n, 16)
    src4 = src * 4
    src4cat = jnp.concatenate([src4, src4 + 1, src4 + 2, src4 + 3])
    s2m = _make_segsum16x4(n_pad, e_pad)(h1lin, src4cat, dst, z16)

    w3p = jnp.pad(W3, ((0, 0), (0, 7)))
    h3hatp = _make_tc3(n, pad)(s2m, h1hat, dinv, W2, b2.reshape(1, h), w3p)

    s3p = segsum8(h3hatp, src, dst, z8)
    out = _make_tc4(n, pad)(s3p, h3hatp, dinv, b3.reshape(1, 1))
    return out


# speedup vs baseline: 32.8047x; 1.0101x over previous
import functools

import jax
import jax.numpy as jnp
from jax import lax
from jax.experimental import pallas as pl
from jax.experimental.pallas import tpu as pltpu
from jax.experimental.pallas import tpu_sc as plsc

NC = 2
NS = 16
ROW_BLK = 1280


def _mesh():
    return plsc.VectorSubcoreMesh(core_axis_name="c", subcore_axis_name="s")


_SC_PARAMS = pltpu.CompilerParams(use_tc_tiling_on_sc=False)


def _make_deg_count(n, e_pad):
    per_tile = e_pad // (NC * NS)
    k = 1024
    assert per_tile % k == 0 and n % (NS * ROW_BLK) == 0
    iters = per_tile // k
    rows_per = n // NS
    n_copies = rows_per // ROW_BLK

    @functools.partial(
        pl.kernel,
        out_type=jax.ShapeDtypeStruct((NC, n, 4), jnp.float32),
        mesh=_mesh(),
        compiler_params=_SC_PARAMS,
        scratch_types=[
            pltpu.VMEM((8, 128), jnp.int32),
            pltpu.VMEM((128, 4), jnp.float32),
            pltpu.VMEM((ROW_BLK, 4), jnp.float32),
            pltpu.VMEM_SHARED((n, 4), jnp.float32),
            pltpu.SemaphoreType.DMA,
        ],
    )
    def deg_kernel(dst_hbm, ones_hbm, zrows, out, dst_v, ones_v, zb, acc, sem):
        c = lax.axis_index("c")
        s = lax.axis_index("s")
        g = c * NS + s
        pltpu.sync_copy(ones_hbm, ones_v)
        pltpu.sync_copy(zrows, zb)
        rbase = s * rows_per
        for z in range(n_copies):
            pltpu.sync_copy(zb, acc.at[pl.ds(rbase + z * ROW_BLK, ROW_BLK)])
        plsc.subcore_barrier()
        row0 = g * (per_tile // 128)

        @pl.loop(0, iters)
        def _(i):
            pltpu.sync_copy(dst_hbm.at[pl.ds(row0 + i * (k // 128), k // 128)], dst_v)
            for j in range(k // 128):
                pltpu.sync_copy(ones_v, acc.at[dst_v.at[j]], add=True)

        plsc.subcore_barrier()
        for z in range(n_copies):
            sl = pl.ds(rbase + z * ROW_BLK, ROW_BLK)
            pltpu.sync_copy(acc.at[sl], out.at[c].at[sl])

    return deg_kernel


def _sweep_pipelined(table2d, src_hbm, dst_hbm, acc, srcb, dstb, rowsb, gsems,
                     ssem, isem, sbase, dbase, iters, k, m):
    def load(it, b):
        pltpu.async_copy(src_hbm.at[pl.ds(sbase + it * k, k)], srcb[b], isem)
        pltpu.async_copy(dst_hbm.at[pl.ds(dbase + it * k, k)], dstb[b], isem)

    def wait_idx(b):
        pltpu.make_async_copy(src_hbm.at[pl.ds(sbase, k)], srcb[b], isem).wait()
        pltpu.make_async_copy(dst_hbm.at[pl.ds(dbase, k)], dstb[b], isem).wait()

    def gather(b):
        pltpu.async_copy(table2d.at[srcb[b]], rowsb[b], gsems[b])

    def wait_gather(b):
        pltpu.make_async_copy(table2d.at[srcb[b]], rowsb[b], gsems[b]).wait()

    def scatter(b):
        pltpu.async_copy(rowsb[b], acc.at[dstb[b]], ssem, add=True).wait()

    load(0, 0)
    wait_idx(0)
    gather(0)
    load(1, 1)

    @pl.loop(0, iters, step=2)
    def _(i):
        wait_idx(1)
        gather(1)
        wait_gather(0)

        @pl.when(i + 2 < iters)
        def _():
            load(i + 2, 0)

        scatter(0)

        @pl.when(i + 2 < iters)
        def _():
            wait_idx(0)
            gather(0)

        wait_gather(1)

        @pl.when(i + 3 < iters)
        def _():
            load(i + 3, 1)

        scatter(1)


def _make_segsum8(n, e_pad):
    per_tile = e_pad // (NC * NS)
    k = 1792
    m = k // 128
    assert per_tile % k == 0 and (per_tile // k) % 2 == 0
    iters = per_tile // k
    rows_per = n // NS
    n_copies = rows_per // ROW_BLK

    @functools.partial(
        pl.kernel,
        out_type=jax.ShapeDtypeStruct((NC, n, 8), jnp.float32),
        mesh=_mesh(),
        compiler_params=_SC_PARAMS,
        scratch_types=[
            pltpu.VMEM((k,), jnp.int32), pltpu.VMEM((k,), jnp.int32),
            pltpu.VMEM((k,), jnp.int32), pltpu.VMEM((k,), jnp.int32),
            pltpu.VMEM((k, 8), jnp.float32), pltpu.VMEM((k, 8), jnp.float32),
            pltpu.VMEM((ROW_BLK, 8), jnp.float32),
            pltpu.VMEM_SHARED((n, 8), jnp.float32),
            pltpu.SemaphoreType.DMA, pltpu.SemaphoreType.DMA,
            pltpu.SemaphoreType.DMA, pltpu.SemaphoreType.DMA,
        ],
    )
    def segsum_kernel(table, src_hbm, dst_hbm, zrows, out,
                      s0, s1, d0, d1, r0, r1, zb, acc, gs0, gs1, ss, isem):
        c = lax.axis_index("c")
        s = lax.axis_index("s")
        g = c * NS + s
        pltpu.sync_copy(zrows, zb)
        rbase = s * rows_per
        for z in range(n_copies):
            pltpu.sync_copy(zb, acc.at[pl.ds(rbase + z * ROW_BLK, ROW_BLK)])
        plsc.subcore_barrier()
        _sweep_pipelined(table, src_hbm, dst_hbm, acc, (s0, s1), (d0, d1),
                         (r0, r1), (gs0, gs1), ss, isem, g * per_tile,
                         g * per_tile, iters, k, m)
        plsc.subcore_barrier()
        for z in range(n_copies):
            sl = pl.ds(rbase + z * ROW_BLK, ROW_BLK)
            pltpu.sync_copy(acc.at[sl], out.at[c].at[sl])

    return segsum_kernel


def _make_segsum16x4(n, e_pad):
    per_tile = e_pad // NS
    k = 512
    m = k // 128
    assert per_tile % k == 0 and (per_tile // k) % 2 == 0
    iters = per_tile // k
    rows_per = n // NS
    row_blk = 320
    n_copies = rows_per // row_blk

    @functools.partial(
        pl.kernel,
        out_type=jax.ShapeDtypeStruct((n, 64), jnp.float32),
        mesh=_mesh(),
        compiler_params=_SC_PARAMS,
        scratch_types=[
            pltpu.VMEM((k,), jnp.int32), pltpu.VMEM((k,), jnp.int32),
            pltpu.VMEM((k,), jnp.int32), pltpu.VMEM((k,), jnp.int32),
            pltpu.VMEM((k, 16), jnp.float32), pltpu.VMEM((k, 16), jnp.float32),
            pltpu.VMEM((row_blk, 16), jnp.float32),
            pltpu.VMEM_SHARED((n, 16), jnp.float32),
            pltpu.SemaphoreType.DMA, pltpu.SemaphoreType.DMA,
            pltpu.SemaphoreType.DMA, pltpu.SemaphoreType.DMA,
        ],
    )
    def segsum_kernel(table, src_hbm, dst_hbm, zrows, out,
                      s0, s1, d0, d1, r0, r1, zb, acc, gs0, gs1, ss, isem):
        c = lax.axis_index("c")
        s = lax.axis_index("s")
        pltpu.sync_copy(zrows, zb)
        rbase = s * rows_per
        ebase = s * per_tile
        for cc in range(2):
            cid = c * 2 + cc
            for z in range(n_copies):
                pltpu.sync_copy(zb, acc.at[pl.ds(rbase + z * row_blk, row_blk)])
            plsc.subcore_barrier()
            _sweep_pipelined(table, src_hbm, dst_hbm, acc, (s0, s1),
                             (d0, d1), (r0, r1), (gs0, gs1), ss, isem,
                             cid * e_pad + ebase, ebase, iters, k, m)
            plsc.subcore_barrier()
            for z in range(n_copies):
                sl = pl.ds(rbase + z * row_blk, row_blk)
                pltpu.sync_copy(acc.at[sl], out.at[sl, pl.ds(cid * 16, 16)])
            plsc.subcore_barrier()

    return segsum_kernel


_R = 2000


def _row0_mask(pid, r):
    return (lax.broadcasted_iota(jnp.int32, (r, 1), 0) == 0) & (pid == 0)


def _make_tc1(n, pad_edges):
    def body(degp, xp, xhat, dinv):
        pid = pl.program_id(0)
        d = degp[...]
        deg = d[0, :, 0:1] + d[1, :, 0:1] + 1.0
        deg = deg - jnp.where(_row0_mask(pid, _R), float(pad_edges), 0.0)
        di = lax.rsqrt(deg)
        dinv[...] = di
        col5 = (lax.broadcasted_iota(jnp.int32, (1, 8), 1) == 5).astype(jnp.float32)
        xhat[...] = xp[...] * di + di * col5

    return pl.pallas_call(
        body,
        grid=(n // _R,),
        in_specs=[
            pl.BlockSpec((NC, _R, 4), lambda i: (0, i, 0)),
            pl.BlockSpec((_R, 8), lambda i: (i, 0)),
        ],
        out_specs=[
            pl.BlockSpec((_R, 8), lambda i: (i, 0)),
            pl.BlockSpec((_R, 1), lambda i: (i, 0)),
        ],
        out_shape=[
            jax.ShapeDtypeStruct((n, 8), jnp.float32),
            jax.ShapeDtypeStruct((n, 1), jnp.float32),
        ],
    )


def _make_tc2(n, pad_edges):
    def body(s1p, xhat, w1, b1, out):
        pid = pl.program_id(0)
        sp = s1p[...]
        xh = xhat[...]
        di = xh[:, 5:6]
        s = sp[0] + sp[1]
        s = s - jnp.where(_row0_mask(pid, _R), float(pad_edges) * xh[0:1, :], 0.0)
        a1 = di * (s + xh)
        h1 = jnp.dot(a1, w1[...], preferred_element_type=jnp.float32,
                     precision=lax.Precision.HIGHEST) + b1[...]
        out[...] = jnp.maximum(h1, 0.0) * di

    return pl.pallas_call(
        body,
        grid=(n // _R,),
        in_specs=[
            pl.BlockSpec((NC, _R, 8), lambda i: (0, i, 0)),
            pl.BlockSpec((_R, 8), lambda i: (i, 0)),
            pl.BlockSpec((8, 64), lambda i: (0, 0)),
            pl.BlockSpec((1, 64), lambda i: (0, 0)),
        ],
        out_specs=pl.BlockSpec((_R, 64), lambda i: (i, 0)),
        out_shape=jax.ShapeDtypeStruct((n, 64), jnp.float32),
    )


def _make_tc3(n, pad_edges):
    def body(s2c, t4, dinv, w2, b2, w3p, out):
        pid = pl.program_id(0)
        hh = t4[...]
        s = s2c[...]
        s = s - jnp.where(_row0_mask(pid, _R), float(pad_edges) * hh[0:1, :], 0.0)
        a2 = dinv[...] * (s + hh)
        h2 = jnp.dot(a2, w2[...], preferred_element_type=jnp.float32,
                     precision=lax.Precision.HIGHEST) + b2[...]
        h2 = jnp.maximum(h2, 0.0)
        t = jnp.dot(h2, w3p[...], preferred_element_type=jnp.float32,
                    precision=lax.Precision.HIGHEST)
        di = dinv[...]
        col1 = (lax.broadcasted_iota(jnp.int32, (1, 8), 1) == 1).astype(jnp.float32)
        out[...] = t * di + di * col1

    return pl.pallas_call(
        body,
        grid=(n // _R,),
        in_specs=[
            pl.BlockSpec((_R, 64), lambda i: (i, 0)),
            pl.BlockSpec((_R, 64), lambda i: (i, 0)),
            pl.BlockSpec((_R, 1), lambda i: (i, 0)),
            pl.BlockSpec((64, 64), lambda i: (0, 0)),
            pl.BlockSpec((1, 64), lambda i: (0, 0)),
            pl.BlockSpec((64, 8), lambda i: (0, 0)),
        ],
        out_specs=pl.BlockSpec((_R, 8), lambda i: (i, 0)),
        out_shape=jax.ShapeDtypeStruct((n, 8), jnp.float32),
    )


def _make_tc4(n, pad_edges):
    def body(s3p, h3hat, b3, out):
        pid = pl.program_id(0)
        sp = s3p[...]
        hh = h3hat[...]
        s = sp[0] + sp[1]
        s = s - jnp.where(_row0_mask(pid, _R), float(pad_edges) * hh[0:1, :], 0.0)
        o = hh[:, 1:2] * (s + hh)
        out[...] = o[:, 0:1] + b3[...]

    return pl.pallas_call(
        body,
        grid=(n // _R,),
        in_specs=[
            pl.BlockSpec((NC, _R, 8), lambda i: (0, i, 0)),
            pl.BlockSpec((_R, 8), lambda i: (i, 0)),
            pl.BlockSpec((1, 1), lambda i: (0, 0)),
        ],
        out_specs=pl.BlockSpec((_R, 1), lambda i: (i, 0)),
        out_shape=jax.ShapeDtypeStruct((n, 1), jnp.float32),
    )


def kernel(x, edge_index, W1, b1, W2, b2, W3, b3):
    n, f = x.shape
    e = edge_index.shape[1]
    h = W1.shape[1]
    assert f == 5 and h == 64 and W3.shape[1] == 1

    unit = NC * NS * 1024
    e_pad = ((e + unit - 1) // unit) * unit
    pad = e_pad - e
    n_unit = NS * ROW_BLK
    n_pad = ((n + n_unit - 1) // n_unit) * n_unit

    src = jnp.concatenate([edge_index[0], jnp.zeros((pad,), jnp.int32)])
    dst = jnp.concatenate([edge_index[1], jnp.zeros((pad,), jnp.int32)])
    dst2d = dst.reshape(-1, 128)
    ones = jnp.ones((128, 4), jnp.float32)
    z4 = jnp.zeros((ROW_BLK, 4), jnp.float32)
    z8 = jnp.zeros((ROW_BLK, 8), jnp.float32)

    degp = _make_deg_count(n_pad, e_pad)(dst2d, ones, z4)
    xpad = jnp.pad(x, ((0, 0), (0, 8 - f)))
    xhat, dinv = _make_tc1(n, pad)(degp, xpad)

    segsum8 = _make_segsum8(n_pad, e_pad)
    s1p = segsum8(xhat, src, dst, z8)
    w1p = jnp.pad(W1, ((0, 8 - f), (0, 0)))
    h1hat = _make_tc2(n, pad)(s1p, xhat, w1p, b1.reshape(1, h))

    z16 = jnp.zeros((320, 16), jnp.float32)
    h1lin = h1hat.reshape(4 * n, 16)
    src4 = src * 4
    src4cat = jnp.concatenate([src4, src4 + 1, src4 + 2, src4 + 3])
    s2m = _make_segsum16x4(n_pad, e_pad)(h1lin, src4cat, dst, z16)

    w3p = jnp.pad(W3, ((0, 0), (0, 7)))
    h3hatp = _make_tc3(n, pad)(s2m, h1hat, dinv, W2, b2.reshape(1, h), w3p)

    s3p = segsum8(h3hatp, src, dst, z8)
    out = _make_tc4(n, pad)(s3p, h3hatp, b3.reshape(1, 1))
    return out


# speedup vs baseline: 35.7794x; 1.0907x over previous
import functools

import jax
import jax.numpy as jnp
from jax import lax
from jax.experimental import pallas as pl
from jax.experimental.pallas import tpu as pltpu
from jax.experimental.pallas import tpu_sc as plsc

NC = 2
NS = 16
ROW_BLK = 1280


def _mesh():
    return plsc.VectorSubcoreMesh(core_axis_name="c", subcore_axis_name="s")


_SC_PARAMS = pltpu.CompilerParams(use_tc_tiling_on_sc=False)


def _make_deg_count(n, e_pad):
    per_tile = e_pad // (NC * NS)
    k = 1024
    assert per_tile % k == 0 and n % (NS * ROW_BLK) == 0
    iters = per_tile // k
    rows_per = n // NS
    n_copies = rows_per // ROW_BLK

    @functools.partial(
        pl.kernel,
        out_type=jax.ShapeDtypeStruct((n, 128), jnp.float32),
        mesh=_mesh(),
        compiler_params=_SC_PARAMS,
        scratch_types=[
            pltpu.VMEM((8, 128), jnp.int32),
            pltpu.VMEM((128, 8), jnp.float32),
            pltpu.VMEM((ROW_BLK, 8), jnp.float32),
            pltpu.VMEM_SHARED((n, 8), jnp.float32),
            pltpu.SemaphoreType.DMA,
        ],
    )
    def deg_kernel(dst_hbm, ones_hbm, zrows, out, dst_v, ones_v, zb, acc, sem):
        c = lax.axis_index("c")
        s = lax.axis_index("s")
        g = c * NS + s
        pltpu.sync_copy(ones_hbm, ones_v)
        pltpu.sync_copy(zrows, zb)
        rbase = s * rows_per
        for z in range(n_copies):
            pltpu.sync_copy(zb, acc.at[pl.ds(rbase + z * ROW_BLK, ROW_BLK)])
        plsc.subcore_barrier()
        row0 = g * (per_tile // 128)

        @pl.loop(0, iters)
        def _(i):
            pltpu.sync_copy(dst_hbm.at[pl.ds(row0 + i * (k // 128), k // 128)], dst_v)
            for j in range(k // 128):
                pltpu.sync_copy(ones_v, acc.at[dst_v.at[j]], add=True)

        plsc.subcore_barrier()
        for z in range(n_copies):
            sl = pl.ds(rbase + z * ROW_BLK, ROW_BLK)
            pltpu.sync_copy(acc.at[sl], out.at[sl, pl.ds(c * 8, 8)])

    return deg_kernel


def _sweep_pipelined(table2d, src_hbm, dst_hbm, acc, srcb, dstb, rowsb, gsems,
                     ssem, isem, sbase, dbase, iters, k, m):
    def load(it, b):
        pltpu.async_copy(src_hbm.at[pl.ds(sbase + it * k, k)], srcb[b], isem)
        pltpu.async_copy(dst_hbm.at[pl.ds(dbase + it * k, k)], dstb[b], isem)

    def wait_idx(b):
        pltpu.make_async_copy(src_hbm.at[pl.ds(sbase, k)], srcb[b], isem).wait()
        pltpu.make_async_copy(dst_hbm.at[pl.ds(dbase, k)], dstb[b], isem).wait()

    def gather(b):
        pltpu.async_copy(table2d.at[srcb[b]], rowsb[b], gsems[b])

    def wait_gather(b):
        pltpu.make_async_copy(table2d.at[srcb[b]], rowsb[b], gsems[b]).wait()

    def scatter(b):
        pltpu.async_copy(rowsb[b], acc.at[dstb[b]], ssem, add=True).wait()

    load(0, 0)
    wait_idx(0)
    gather(0)
    load(1, 1)

    @pl.loop(0, iters, step=2)
    def _(i):
        wait_idx(1)
        gather(1)
        wait_gather(0)

        @pl.when(i + 2 < iters)
        def _():
            load(i + 2, 0)

        scatter(0)

        @pl.when(i + 2 < iters)
        def _():
            wait_idx(0)
            gather(0)

        wait_gather(1)

        @pl.when(i + 3 < iters)
        def _():
            load(i + 3, 1)

        scatter(1)


def _make_segsum8(n, e_pad):
    per_tile = e_pad // (NC * NS)
    k = 1792
    m = k // 128
    assert per_tile % k == 0 and (per_tile // k) % 2 == 0
    iters = per_tile // k
    rows_per = n // NS
    n_copies = rows_per // ROW_BLK

    @functools.partial(
        pl.kernel,
        out_type=jax.ShapeDtypeStruct((n, 128), jnp.float32),
        mesh=_mesh(),
        compiler_params=_SC_PARAMS,
        scratch_types=[
            pltpu.VMEM((k,), jnp.int32), pltpu.VMEM((k,), jnp.int32),
            pltpu.VMEM((k,), jnp.int32), pltpu.VMEM((k,), jnp.int32),
            pltpu.VMEM((k, 8), jnp.float32), pltpu.VMEM((k, 8), jnp.float32),
            pltpu.VMEM((ROW_BLK, 8), jnp.float32),
            pltpu.VMEM_SHARED((n, 8), jnp.float32),
            pltpu.SemaphoreType.DMA, pltpu.SemaphoreType.DMA,
            pltpu.SemaphoreType.DMA, pltpu.SemaphoreType.DMA,
        ],
    )
    def segsum_kernel(table, src_hbm, dst_hbm, zrows, out,
                      s0, s1, d0, d1, r0, r1, zb, acc, gs0, gs1, ss, isem):
        c = lax.axis_index("c")
        s = lax.axis_index("s")
        g = c * NS + s
        pltpu.sync_copy(zrows, zb)
        rbase = s * rows_per
        for z in range(n_copies):
            pltpu.sync_copy(zb, acc.at[pl.ds(rbase + z * ROW_BLK, ROW_BLK)])
        plsc.subcore_barrier()
        _sweep_pipelined(table, src_hbm, dst_hbm, acc, (s0, s1), (d0, d1),
                         (r0, r1), (gs0, gs1), ss, isem, g * per_tile,
                         g * per_tile, iters, k, m)
        plsc.subcore_barrier()
        for z in range(n_copies):
            sl = pl.ds(rbase + z * ROW_BLK, ROW_BLK)
            pltpu.sync_copy(acc.at[sl], out.at[sl, pl.ds(c * 8, 8)])

    return segsum_kernel


def _make_segsum16x4(n, e_pad):
    per_tile = e_pad // NS
    k = 512
    m = k // 128
    assert per_tile % k == 0 and (per_tile // k) % 2 == 0
    iters = per_tile // k
    rows_per = n // NS
    row_blk = 320
    n_copies = rows_per // row_blk

    @functools.partial(
        pl.kernel,
        out_type=jax.ShapeDtypeStruct((n, 64), jnp.float32),
        mesh=_mesh(),
        compiler_params=_SC_PARAMS,
        scratch_types=[
            pltpu.VMEM((k,), jnp.int32), pltpu.VMEM((k,), jnp.int32),
            pltpu.VMEM((k,), jnp.int32), pltpu.VMEM((k,), jnp.int32),
            pltpu.VMEM((k, 16), jnp.float32), pltpu.VMEM((k, 16), jnp.float32),
            pltpu.VMEM((row_blk, 16), jnp.float32),
            pltpu.VMEM_SHARED((n, 16), jnp.float32),
            pltpu.SemaphoreType.DMA, pltpu.SemaphoreType.DMA,
            pltpu.SemaphoreType.DMA, pltpu.SemaphoreType.DMA,
        ],
    )
    def segsum_kernel(table, src_hbm, dst_hbm, zrows, out,
                      s0, s1, d0, d1, r0, r1, zb, acc, gs0, gs1, ss, isem):
        c = lax.axis_index("c")
        s = lax.axis_index("s")
        pltpu.sync_copy(zrows, zb)
        rbase = s * rows_per
        ebase = s * per_tile
        for cc in range(2):
            cid = c * 2 + cc
            for z in range(n_copies):
                pltpu.sync_copy(zb, acc.at[pl.ds(rbase + z * row_blk, row_blk)])
            plsc.subcore_barrier()
            _sweep_pipelined(table, src_hbm, dst_hbm, acc, (s0, s1),
                             (d0, d1), (r0, r1), (gs0, gs1), ss, isem,
                             cid * e_pad + ebase, ebase, iters, k, m)
            plsc.subcore_barrier()
            for z in range(n_copies):
                sl = pl.ds(rbase + z * row_blk, row_blk)
                pltpu.sync_copy(acc.at[sl], out.at[sl, pl.ds(cid * 16, 16)])
            plsc.subcore_barrier()

    return segsum_kernel


_R = 2000


def _row0_mask(pid, r):
    return (lax.broadcasted_iota(jnp.int32, (r, 1), 0) == 0) & (pid == 0)


def _make_tc1(n, pad_edges):
    def body(degp, xp, xhat, dinv):
        pid = pl.program_id(0)
        d = degp[...]
        deg = d[:, 0:1] + d[:, 8:9] + 1.0
        deg = deg - jnp.where(_row0_mask(pid, _R), float(pad_edges), 0.0)
        di = lax.rsqrt(deg)
        dinv[...] = di
        col5 = (lax.broadcasted_iota(jnp.int32, (1, 8), 1) == 5).astype(jnp.float32)
        xhat[...] = xp[...] * di + di * col5

    return pl.pallas_call(
        body,
        grid=(n // _R,),
        in_specs=[
            pl.BlockSpec((_R, 128), lambda i: (i, 0)),
            pl.BlockSpec((_R, 8), lambda i: (i, 0)),
        ],
        out_specs=[
            pl.BlockSpec((_R, 8), lambda i: (i, 0)),
            pl.BlockSpec((_R, 1), lambda i: (i, 0)),
        ],
        out_shape=[
            jax.ShapeDtypeStruct((n, 8), jnp.float32),
            jax.ShapeDtypeStruct((n, 1), jnp.float32),
        ],
    )


def _make_tc2(n, pad_edges):
    def body(s1p, xhat, w1, b1, out):
        pid = pl.program_id(0)
        sp = s1p[...]
        xh = xhat[...]
        di = xh[:, 5:6]
        s = sp[:, 0:8] + sp[:, 8:16]
        s = s - jnp.where(_row0_mask(pid, _R), float(pad_edges) * xh[0:1, :], 0.0)
        a1 = di * (s + xh)
        h1 = jnp.dot(a1, w1[...], preferred_element_type=jnp.float32,
                     precision=lax.Precision.HIGHEST) + b1[...]
        out[...] = jnp.maximum(h1, 0.0) * di

    return pl.pallas_call(
        body,
        grid=(n // _R,),
        in_specs=[
            pl.BlockSpec((_R, 128), lambda i: (i, 0)),
            pl.BlockSpec((_R, 8), lambda i: (i, 0)),
            pl.BlockSpec((8, 64), lambda i: (0, 0)),
            pl.BlockSpec((1, 64), lambda i: (0, 0)),
        ],
        out_specs=pl.BlockSpec((_R, 64), lambda i: (i, 0)),
        out_shape=jax.ShapeDtypeStruct((n, 64), jnp.float32),
    )


def _make_tc3(n, pad_edges):
    def body(s2c, t4, dinv, w2, b2, w3p, out):
        pid = pl.program_id(0)
        hh = t4[...]
        s = s2c[...]
        s = s - jnp.where(_row0_mask(pid, _R), float(pad_edges) * hh[0:1, :], 0.0)
        a2 = dinv[...] * (s + hh)
        h2 = jnp.dot(a2, w2[...], preferred_element_type=jnp.float32,
                     precision=lax.Precision.HIGHEST) + b2[...]
        h2 = jnp.maximum(h2, 0.0)
        t = jnp.dot(h2, w3p[...], preferred_element_type=jnp.float32,
                    precision=lax.Precision.HIGHEST)
        di = dinv[...]
        col1 = (lax.broadcasted_iota(jnp.int32, (1, 8), 1) == 1).astype(jnp.float32)
        out[...] = t * di + di * col1

    return pl.pallas_call(
        body,
        grid=(n // _R,),
        in_specs=[
            pl.BlockSpec((_R, 64), lambda i: (i, 0)),
            pl.BlockSpec((_R, 64), lambda i: (i, 0)),
            pl.BlockSpec((_R, 1), lambda i: (i, 0)),
            pl.BlockSpec((64, 64), lambda i: (0, 0)),
            pl.BlockSpec((1, 64), lambda i: (0, 0)),
            pl.BlockSpec((64, 8), lambda i: (0, 0)),
        ],
        out_specs=pl.BlockSpec((_R, 8), lambda i: (i, 0)),
        out_shape=jax.ShapeDtypeStruct((n, 8), jnp.float32),
    )


def _make_tc4(n, pad_edges):
    def body(s3p, h3hat, b3, out):
        pid = pl.program_id(0)
        sp = s3p[...]
        hh = h3hat[...]
        s = sp[:, 0:8] + sp[:, 8:16]
        s = s - jnp.where(_row0_mask(pid, _R), float(pad_edges) * hh[0:1, :], 0.0)
        o = hh[:, 1:2] * (s + hh)
        out[...] = o[:, 0:1] + b3[...]

    return pl.pallas_call(
        body,
        grid=(n // _R,),
        in_specs=[
            pl.BlockSpec((_R, 128), lambda i: (i, 0)),
            pl.BlockSpec((_R, 8), lambda i: (i, 0)),
            pl.BlockSpec((1, 1), lambda i: (0, 0)),
        ],
        out_specs=pl.BlockSpec((_R, 1), lambda i: (i, 0)),
        out_shape=jax.ShapeDtypeStruct((n, 1), jnp.float32),
    )


def kernel(x, edge_index, W1, b1, W2, b2, W3, b3):
    n, f = x.shape
    e = edge_index.shape[1]
    h = W1.shape[1]
    assert f == 5 and h == 64 and W3.shape[1] == 1

    unit = NC * NS * 1024
    e_pad = ((e + unit - 1) // unit) * unit
    pad = e_pad - e
    n_unit = NS * ROW_BLK
    n_pad = ((n + n_unit - 1) // n_unit) * n_unit

    src = jnp.concatenate([edge_index[0], jnp.zeros((pad,), jnp.int32)])
    dst = jnp.concatenate([edge_index[1], jnp.zeros((pad,), jnp.int32)])
    dst2d = dst.reshape(-1, 128)
    ones = jnp.ones((128, 8), jnp.float32)
    z4 = jnp.zeros((ROW_BLK, 8), jnp.float32)
    z8 = jnp.zeros((ROW_BLK, 8), jnp.float32)

    degp = _make_deg_count(n_pad, e_pad)(dst2d, ones, z4)
    xpad = jnp.pad(x, ((0, 0), (0, 8 - f)))
    xhat, dinv = _make_tc1(n, pad)(degp, xpad)

    segsum8 = _make_segsum8(n_pad, e_pad)
    s1p = segsum8(xhat, src, dst, z8)
    w1p = jnp.pad(W1, ((0, 8 - f), (0, 0)))
    h1hat = _make_tc2(n, pad)(s1p, xhat, w1p, b1.reshape(1, h))

    z16 = jnp.zeros((320, 16), jnp.float32)
    h1lin = h1hat.reshape(4 * n, 16)
    src4 = src * 4
    src4cat = jnp.concatenate([src4, src4 + 1, src4 + 2, src4 + 3])
    s2m = _make_segsum16x4(n_pad, e_pad)(h1lin, src4cat, dst, z16)

    w3p = jnp.pad(W3, ((0, 0), (0, 7)))
    h3hatp = _make_tc3(n, pad)(s2m, h1hat, dinv, W2, b2.reshape(1, h), w3p)

    s3p = segsum8(h3hatp, src, dst, z8)
    out = _make_tc4(n, pad)(s3p, h3hatp, b3.reshape(1, 1))
    return out
